# baseline (speedup 1.0000x reference)
import functools

import jax
import jax.numpy as jnp
from jax import lax
from jax.experimental import pallas as pl
from jax.experimental.pallas import tpu as pltpu
from jax.experimental.pallas import tpu_sc as plsc

B = 16384
D = 64
DP = 128
NC = 2
NS = 16
NW = NC * NS
BPW = B // NW
L = 16
USER_N = 100000
ITEM_N = 1000000
IT_FULL = ITEM_N // DP
IT_B = IT_FULL + 1
US_FULL = USER_N // DP
US_B = US_FULL + 1
SCR = B + NW
SEG = 4096
NSEG = B // SEG
FB = 64

_i32 = jnp.int32


def _phase1(u_h, i_h, j_h, utabT, itabT, utailT, itailT, su, si, sj,
            chunk, hist_it, off_it, cur_it, hist_u, off_u, cur_u,
            ent_it, ent_u, slab, fbu, fbi, fbj, du, di, dj, fsem):
    wid = lax.axis_index("s") * NC + lax.axis_index("c")
    dummy = B + wid
    lane = lax.iota(_i32, L)
    lane0 = lane == 0
    ones = jnp.ones((L,), _i32)

    it_lo = (wid * IT_B) // NW
    it_hi = ((wid + 1) * IT_B) // NW
    us_lo = (wid * US_B) // NW
    us_hi = ((wid + 1) * US_B) // NW

    def zero(ref, n16):
        for t in range(n16):
            ref[pl.ds(t * L, L)] = jnp.zeros((L,), _i32)

    def fill(ref, n16, val):
        for t in range(n16):
            ref[pl.ds(t * L, L)] = jnp.full((L,), val, _i32)

    zero(hist_it, 16)
    zero(hist_u, 2)
    fill(du, FB // L, dummy)
    fill(di, FB // L, dummy)
    fill(dj, FB // L, dummy)

    def scan_hist(src_h, hist, lo, hi):
        for seg in range(NSEG):
            pltpu.sync_copy(src_h.at[pl.ds(seg * SEG, SEG)], chunk)

            def vec(t, carry):
                v = chunk[pl.ds(t * L, L)]
                blk = lax.shift_right_logical(v, 7)
                m = (blk >= lo) & (blk < hi)
                plsc.addupdate_scatter(
                    hist, [jnp.where(m, blk - lo, 0)], ones, mask=m)
                return carry

            lax.fori_loop(0, SEG // L, vec, 0)

    scan_hist(u_h, hist_u, us_lo, us_hi)
    scan_hist(i_h, hist_it, it_lo, it_hi)
    scan_hist(j_h, hist_it, it_lo, it_hi)

    def prefix(hist, off, cur, n16):
        run = _i32(0)
        for t in range(n16):
            v = hist[pl.ds(t * L, L)]
            cs = plsc.cumsum(v)
            ex = cs - v + run
            off[pl.ds(t * L, L)] = ex
            cur[pl.ds(t * L, L)] = ex
            run = run + cs[15]

    prefix(hist_it, off_it, cur_it, 16)
    prefix(hist_u, off_u, cur_u, 2)

    def scan_emit(src_h, ent, cur, lo, hi, tag):
        for seg in range(NSEG):
            pltpu.sync_copy(src_h.at[pl.ds(seg * SEG, SEG)], chunk)

            def vec(t, carry):
                b0 = seg * SEG + t * L
                v = chunk[pl.ds(t * L, L)]
                blk = lax.shift_right_logical(v, 7)
                m0 = (blk >= lo) & (blk < hi)

                def cond(st):
                    return jnp.any(st[0])

                def body(st):
                    mm = st[0]
                    l = plsc.all_reduce_ffs(mm)[0]
                    sval = jnp.sum(jnp.where(lane == l, v, 0))
                    rel = lax.shift_right_logical(sval, 7) - lo
                    relv = jnp.full((L,), rel, _i32)
                    slot = plsc.load_gather(cur, [relv])[0]
                    entry = ((b0 + l) << 8) | (tag << 7) | (sval & 127)
                    plsc.store_scatter(ent, [jnp.full((L,), slot, _i32)],
                                       jnp.full((L,), entry, _i32), mask=lane0)
                    plsc.store_scatter(cur, [relv],
                                       jnp.full((L,), slot + 1, _i32),
                                       mask=lane0)
                    return (mm & (lane != l),)

                lax.while_loop(cond, body, (m0,))
                return carry

            lax.fori_loop(0, SEG // L, vec, 0)

    scan_emit(u_h, ent_u, cur_u, us_lo, us_hi, 0)
    scan_emit(i_h, ent_it, cur_it, it_lo, it_hi, 0)
    scan_emit(j_h, ent_it, cur_it, it_lo, it_hi, 1)

    def flush(fbref, dref, tgt):
        pltpu.async_copy(fbref, tgt.at[dref], fsem).wait()
        fill(dref, FB // L, dummy)

    def append(fbref, dref, tgt, cols, b, fc):
        for c in range(4):
            fbref[fc, pl.ds(c * L, L)] = cols[c]
        plsc.store_scatter(dref, [jnp.full((L,), fc, _i32)],
                           jnp.full((L,), b, _i32), mask=lane0)
        fc = fc + 1

        def do_flush(_):
            flush(fbref, dref, tgt)
            return _i32(0)

        return lax.cond(fc == FB, do_flush, lambda _: fc, 0)

    def extract_user(t, fcu):
        blk = us_lo + t

        def cp_tail(_):
            pltpu.sync_copy(utailT, slab)
            return 0

        def cp_full(_):
            pltpu.sync_copy(utabT.at[:, pl.ds(blk * DP, DP)], slab)
            return 0

        lax.cond(blk == US_FULL, cp_tail, cp_full, 0)
        s = plsc.load_gather(off_u, [jnp.full((L,), t, _i32)])[0]
        e = plsc.load_gather(off_u, [jnp.full((L,), t + 1, _i32)])[0]

        def ent_loop(ei, fc):
            ev = ent_u[pl.ds(ei, L)][0]
            b = lax.shift_right_logical(ev, 8)
            lnv = jnp.full((L,), ev & 127, _i32)
            cols = [plsc.load_gather(slab, [c + lane, lnv])
                    for c in (0, 16, 32, 48)]
            return append(fbu, du, su, cols, b, fc)

        return lax.fori_loop(s, e, ent_loop, fcu)

    fcu = lax.fori_loop(0, us_hi - us_lo, extract_user, _i32(0))

    def extract_item(t, fcs):
        blk = it_lo + t

        def cp_tail(_):
            pltpu.sync_copy(itailT, slab)
            return 0

        def cp_full(_):
            pltpu.sync_copy(itabT.at[:, pl.ds(blk * DP, DP)], slab)
            return 0

        lax.cond(blk == IT_FULL, cp_tail, cp_full, 0)
        s = plsc.load_gather(off_it, [jnp.full((L,), t, _i32)])[0]
        e = plsc.load_gather(off_it, [jnp.full((L,), t + 1, _i32)])[0]

        def ent_loop(ei, fc):
            fci, fcj = fc
            ev = ent_it[pl.ds(ei, L)][0]
            b = lax.shift_right_logical(ev, 8)
            tg = lax.shift_right_logical(ev, 7) & 1
            lnv = jnp.full((L,), ev & 127, _i32)
            cols = [plsc.load_gather(slab, [c + lane, lnv])
                    for c in (0, 16, 32, 48)]
            return lax.cond(
                tg == 1,
                lambda _: (fci, append(fbj, dj, sj, cols, b, fcj)),
                lambda _: (append(fbi, di, si, cols, b, fci), fcj),
                0)

        return lax.fori_loop(s, e, ent_loop, fcs)

    lax.fori_loop(0, it_hi - it_lo, extract_item, (_i32(0), _i32(0)))

    flush(fbu, du, su)
    flush(fbi, di, si)
    flush(fbj, dj, sj)


def _phase2(su, si, sj, out_i, out_j, ur, vir, vjr, oi, oj, sem):
    wid = lax.axis_index("s") * NC + lax.axis_index("c")
    b0 = wid * BPW
    lane = lax.iota(_i32, L)
    last = lane == (L - 1)

    for k in range(BPW // DP):
        pltpu.sync_copy(su.at[pl.ds(b0 + k * DP, DP)], ur)
        pltpu.sync_copy(si.at[pl.ds(b0 + k * DP, DP)], vir)
        pltpu.sync_copy(sj.at[pl.ds(b0 + k * DP, DP)], vjr)

        def row(r, carry):
            acc_i = jnp.zeros((L,), jnp.float32)
            acc_j = jnp.zeros((L,), jnp.float32)
            for cc in range(D // L):
                u = ur[r, pl.ds(cc * L, L)]
                vi = vir[r, pl.ds(cc * L, L)]
                vj = vjr[r, pl.ds(cc * L, L)]
                acc_i = acc_i + u * vi
                acc_j = acc_j + u * vj
            gidx = jnp.full((L,), k * DP + r, _i32)
            plsc.store_scatter(oi, [gidx], plsc.cumsum(acc_i), mask=last)
            plsc.store_scatter(oj, [gidx], plsc.cumsum(acc_j), mask=last)
            return carry

        lax.fori_loop(0, DP, row, 0)

    pltpu.sync_copy(oi, out_i.at[pl.ds(b0, BPW)])
    pltpu.sync_copy(oj, out_j.at[pl.ds(b0, BPW)])


def kernel(user, item_i, item_j, embed_user_weight, embed_item_weight):
    mesh = plsc.VectorSubcoreMesh(core_axis_name="c", subcore_axis_name="s")
    cp = pltpu.CompilerParams(
        needs_layout_passes=False, use_tc_tiling_on_sc=True)

    run1 = pl.kernel(
        _phase1,
        mesh=mesh,
        compiler_params=cp,
        out_type=(
            jax.ShapeDtypeStruct((SCR, DP), jnp.float32),
            jax.ShapeDtypeStruct((SCR, DP), jnp.float32),
            jax.ShapeDtypeStruct((SCR, DP), jnp.float32),
        ),
        scratch_types=[
            pltpu.VMEM((SEG,), _i32),
            pltpu.VMEM((256,), _i32),
            pltpu.VMEM((256,), _i32),
            pltpu.VMEM((256,), _i32),
            pltpu.VMEM((32,), _i32),
            pltpu.VMEM((32,), _i32),
            pltpu.VMEM((32,), _i32),
            pltpu.VMEM((2 * B + L,), _i32),
            pltpu.VMEM((B + L,), _i32),
            pltpu.VMEM((64, DP), jnp.float32),
            pltpu.VMEM((FB, DP), jnp.float32),
            pltpu.VMEM((FB, DP), jnp.float32),
            pltpu.VMEM((FB, DP), jnp.float32),
            pltpu.VMEM((FB,), _i32),
            pltpu.VMEM((FB,), _i32),
            pltpu.VMEM((FB,), _i32),
            pltpu.SemaphoreType.DMA,
        ],
    )

    run2 = pl.kernel(
        _phase2,
        mesh=mesh,
        compiler_params=cp,
        out_type=(
            jax.ShapeDtypeStruct((B,), jnp.float32),
            jax.ShapeDtypeStruct((B,), jnp.float32),
        ),
        scratch_types=[
            pltpu.VMEM((DP, DP), jnp.float32),
            pltpu.VMEM((DP, DP), jnp.float32),
            pltpu.VMEM((DP, DP), jnp.float32),
            pltpu.VMEM((BPW,), jnp.float32),
            pltpu.VMEM((BPW,), jnp.float32),
            pltpu.SemaphoreType.DMA,
        ],
    )

    u = user.astype(_i32)
    i = item_i.astype(_i32)
    j = item_j.astype(_i32)
    utabT = embed_user_weight.T
    itabT = embed_item_weight.T
    utailT = jnp.pad(embed_user_weight[US_FULL * DP:],
                     ((0, DP - (USER_N - US_FULL * DP)), (0, 0))).T
    itailT = jnp.pad(embed_item_weight[IT_FULL * DP:],
                     ((0, DP - (ITEM_N - IT_FULL * DP)), (0, 0))).T
    su, si, sj = run1(u, i, j, utabT, itabT, utailT, itailT)
    return run2(su, si, sj)


# speedup vs baseline: 1.1678x; 1.1678x over previous
import functools

import jax
import jax.numpy as jnp
from jax import lax
from jax.experimental import pallas as pl
from jax.experimental.pallas import tpu as pltpu
from jax.experimental.pallas import tpu_sc as plsc

B = 16384
D = 64
DP = 128
NC = 2
NS = 16
NW = NC * NS
BPW = B // NW
L = 16
USER_N = 100000
ITEM_N = 1000000
IT_FULL = ITEM_N // DP
IT_B = IT_FULL + 1
US_FULL = USER_N // DP
US_B = US_FULL + 1
SCR = B + NW
SEG = 4096
NSEG = B // SEG
FB = 64

_i32 = jnp.int32


def _phase1(u_h, i_h, j_h, utabT, itabT, utailT, itailT, su, si, sj,
            chunk, hist_it, off_it, cur_it, hist_u, off_u, cur_u,
            ent_it, ent_u, slab, fbu, fbi, fbj, du, di, dj, fsem):
    wid = lax.axis_index("s") * NC + lax.axis_index("c")
    dummy = B + wid
    lane = lax.iota(_i32, L)
    lane0 = lane == 0
    ones = jnp.ones((L,), _i32)

    it_lo = (wid * IT_B) // NW
    it_hi = ((wid + 1) * IT_B) // NW
    us_lo = (wid * US_B) // NW
    us_hi = ((wid + 1) * US_B) // NW

    def zero(ref, n16):
        for t in range(n16):
            ref[pl.ds(t * L, L)] = jnp.zeros((L,), _i32)

    def fill(ref, n16, val):
        for t in range(n16):
            ref[pl.ds(t * L, L)] = jnp.full((L,), val, _i32)

    zero(hist_it, 16)
    zero(hist_u, 2)
    fill(du, FB // L, dummy)
    fill(di, FB // L, dummy)
    fill(dj, FB // L, dummy)

    UNR = 8

    def scan_hist(src_h, hist, lo, hi):
        for seg in range(NSEG):
            pltpu.sync_copy(src_h.at[pl.ds(seg * SEG, SEG)], chunk)

            def vec(t, carry):
                for s in range(UNR):
                    v = chunk[pl.ds(t * L * UNR + s * L, L)]
                    blk = lax.shift_right_logical(v, 7)
                    m = (blk >= lo) & (blk < hi)
                    plsc.addupdate_scatter(
                        hist, [jnp.where(m, blk - lo, 0)], ones, mask=m)
                return carry

            lax.fori_loop(0, SEG // L // UNR, vec, 0)

    scan_hist(u_h, hist_u, us_lo, us_hi)
    scan_hist(i_h, hist_it, it_lo, it_hi)
    scan_hist(j_h, hist_it, it_lo, it_hi)

    def prefix(hist, off, cur, n16):
        run = _i32(0)
        for t in range(n16):
            v = hist[pl.ds(t * L, L)]
            cs = plsc.cumsum(v)
            ex = cs - v + run
            off[pl.ds(t * L, L)] = ex
            cur[pl.ds(t * L, L)] = ex
            run = run + cs[15]

    prefix(hist_it, off_it, cur_it, 16)
    prefix(hist_u, off_u, cur_u, 2)

    rank_base = plsc.scan_count(jnp.zeros((L,), _i32))[0][0]

    def scan_emit(src_h, ent, cur, lo, hi, tag):
        tagv = jnp.full((L,), tag << 7, _i32)
        for seg in range(NSEG):
            pltpu.sync_copy(src_h.at[pl.ds(seg * SEG, SEG)], chunk)

            def vec(t, carry):
                for s in range(UNR):
                    b0 = seg * SEG + t * L * UNR + s * L
                    v = chunk[pl.ds(t * L * UNR + s * L, L)]
                    blk = lax.shift_right_logical(v, 7)
                    m = (blk >= lo) & (blk < hi)
                    rank, _ = plsc.scan_count(blk, m)
                    relv = jnp.where(m, blk - lo, 0)
                    base = plsc.load_gather(cur, [relv])
                    slot = base + rank - rank_base
                    entry = ((b0 + lane) << 8) | tagv | (v & 127)
                    plsc.store_scatter(ent, [slot], entry, mask=m)
                    plsc.addupdate_scatter(cur, [relv], ones, mask=m)
                return carry

            lax.fori_loop(0, SEG // L // UNR, vec, 0)

    scan_emit(u_h, ent_u, cur_u, us_lo, us_hi, 0)
    scan_emit(i_h, ent_it, cur_it, it_lo, it_hi, 0)
    scan_emit(j_h, ent_it, cur_it, it_lo, it_hi, 1)

    def flush(fbref, dref, tgt):
        pltpu.async_copy(fbref, tgt.at[dref], fsem).wait()
        fill(dref, FB // L, dummy)

    def append(fbref, dref, tgt, cols, b, fc):
        for c in range(4):
            fbref[fc, pl.ds(c * L, L)] = cols[c]
        plsc.store_scatter(dref, [jnp.full((L,), fc, _i32)],
                           jnp.full((L,), b, _i32), mask=lane0)
        fc = fc + 1

        def do_flush(_):
            flush(fbref, dref, tgt)
            return _i32(0)

        return lax.cond(fc == FB, do_flush, lambda _: fc, 0)

    def extract_user(t, fcu):
        blk = us_lo + t

        def cp_tail(_):
            pltpu.sync_copy(utailT, slab)
            return 0

        def cp_full(_):
            pltpu.sync_copy(utabT.at[:, pl.ds(blk * DP, DP)], slab)
            return 0

        lax.cond(blk == US_FULL, cp_tail, cp_full, 0)
        s = plsc.load_gather(off_u, [jnp.full((L,), t, _i32)])[0]
        e = plsc.load_gather(off_u, [jnp.full((L,), t + 1, _i32)])[0]

        def ent_loop(ei, fc):
            ev = ent_u[pl.ds(ei, L)][0]
            b = lax.shift_right_logical(ev, 8)
            lnv = jnp.full((L,), ev & 127, _i32)
            cols = [plsc.load_gather(slab, [c + lane, lnv])
                    for c in (0, 16, 32, 48)]
            return append(fbu, du, su, cols, b, fc)

        return lax.fori_loop(s, e, ent_loop, fcu)

    fcu = lax.fori_loop(0, us_hi - us_lo, extract_user, _i32(0))

    def extract_item(t, fcs):
        blk = it_lo + t

        def cp_tail(_):
            pltpu.sync_copy(itailT, slab)
            return 0

        def cp_full(_):
            pltpu.sync_copy(itabT.at[:, pl.ds(blk * DP, DP)], slab)
            return 0

        lax.cond(blk == IT_FULL, cp_tail, cp_full, 0)
        s = plsc.load_gather(off_it, [jnp.full((L,), t, _i32)])[0]
        e = plsc.load_gather(off_it, [jnp.full((L,), t + 1, _i32)])[0]

        def ent_loop(ei, fc):
            fci, fcj = fc
            ev = ent_it[pl.ds(ei, L)][0]
            b = lax.shift_right_logical(ev, 8)
            tg = lax.shift_right_logical(ev, 7) & 1
            lnv = jnp.full((L,), ev & 127, _i32)
            cols = [plsc.load_gather(slab, [c + lane, lnv])
                    for c in (0, 16, 32, 48)]
            return lax.cond(
                tg == 1,
                lambda _: (fci, append(fbj, dj, sj, cols, b, fcj)),
                lambda _: (append(fbi, di, si, cols, b, fci), fcj),
                0)

        return lax.fori_loop(s, e, ent_loop, fcs)

    lax.fori_loop(0, it_hi - it_lo, extract_item, (_i32(0), _i32(0)))

    flush(fbu, du, su)
    flush(fbi, di, si)
    flush(fbj, dj, sj)


def _phase2(su, si, sj, out_i, out_j, ur, vir, vjr, oi, oj, sem):
    wid = lax.axis_index("s") * NC + lax.axis_index("c")
    b0 = wid * BPW
    lane = lax.iota(_i32, L)
    last = lane == (L - 1)

    for k in range(BPW // DP):
        pltpu.sync_copy(su.at[pl.ds(b0 + k * DP, DP)], ur)
        pltpu.sync_copy(si.at[pl.ds(b0 + k * DP, DP)], vir)
        pltpu.sync_copy(sj.at[pl.ds(b0 + k * DP, DP)], vjr)

        def row(r, carry):
            acc_i = jnp.zeros((L,), jnp.float32)
            acc_j = jnp.zeros((L,), jnp.float32)
            for cc in range(D // L):
                u = ur[r, pl.ds(cc * L, L)]
                vi = vir[r, pl.ds(cc * L, L)]
                vj = vjr[r, pl.ds(cc * L, L)]
                acc_i = acc_i + u * vi
                acc_j = acc_j + u * vj
            gidx = jnp.full((L,), k * DP + r, _i32)
            plsc.store_scatter(oi, [gidx], plsc.cumsum(acc_i), mask=last)
            plsc.store_scatter(oj, [gidx], plsc.cumsum(acc_j), mask=last)
            return carry

        lax.fori_loop(0, DP, row, 0)

    pltpu.sync_copy(oi, out_i.at[pl.ds(b0, BPW)])
    pltpu.sync_copy(oj, out_j.at[pl.ds(b0, BPW)])


def kernel(user, item_i, item_j, embed_user_weight, embed_item_weight):
    mesh = plsc.VectorSubcoreMesh(core_axis_name="c", subcore_axis_name="s")
    cp = pltpu.CompilerParams(
        needs_layout_passes=False, use_tc_tiling_on_sc=True)

    run1 = pl.kernel(
        _phase1,
        mesh=mesh,
        compiler_params=cp,
        out_type=(
            jax.ShapeDtypeStruct((SCR, DP), jnp.float32),
            jax.ShapeDtypeStruct((SCR, DP), jnp.float32),
            jax.ShapeDtypeStruct((SCR, DP), jnp.float32),
        ),
        scratch_types=[
            pltpu.VMEM((SEG,), _i32),
            pltpu.VMEM((256,), _i32),
            pltpu.VMEM((256,), _i32),
            pltpu.VMEM((256,), _i32),
            pltpu.VMEM((32,), _i32),
            pltpu.VMEM((32,), _i32),
            pltpu.VMEM((32,), _i32),
            pltpu.VMEM((2 * B + L,), _i32),
            pltpu.VMEM((B + L,), _i32),
            pltpu.VMEM((64, DP), jnp.float32),
            pltpu.VMEM((FB, DP), jnp.float32),
            pltpu.VMEM((FB, DP), jnp.float32),
            pltpu.VMEM((FB, DP), jnp.float32),
            pltpu.VMEM((FB,), _i32),
            pltpu.VMEM((FB,), _i32),
            pltpu.VMEM((FB,), _i32),
            pltpu.SemaphoreType.DMA,
        ],
    )

    run2 = pl.kernel(
        _phase2,
        mesh=mesh,
        compiler_params=cp,
        out_type=(
            jax.ShapeDtypeStruct((B,), jnp.float32),
            jax.ShapeDtypeStruct((B,), jnp.float32),
        ),
        scratch_types=[
            pltpu.VMEM((DP, DP), jnp.float32),
            pltpu.VMEM((DP, DP), jnp.float32),
            pltpu.VMEM((DP, DP), jnp.float32),
            pltpu.VMEM((BPW,), jnp.float32),
            pltpu.VMEM((BPW,), jnp.float32),
            pltpu.SemaphoreType.DMA,
        ],
    )

    u = user.astype(_i32)
    i = item_i.astype(_i32)
    j = item_j.astype(_i32)
    utabT = embed_user_weight.T
    itabT = embed_item_weight.T
    utailT = jnp.pad(embed_user_weight[US_FULL * DP:],
                     ((0, DP - (USER_N - US_FULL * DP)), (0, 0))).T
    itailT = jnp.pad(embed_item_weight[IT_FULL * DP:],
                     ((0, DP - (ITEM_N - IT_FULL * DP)), (0, 0))).T
    su, si, sj = run1(u, i, j, utabT, itabT, utailT, itailT)
    return run2(su, si, sj)


# speedup vs baseline: 1.2436x; 1.0650x over previous
import functools

import jax
import jax.numpy as jnp
from jax import lax
from jax.experimental import pallas as pl
from jax.experimental.pallas import tpu as pltpu
from jax.experimental.pallas import tpu_sc as plsc

B = 16384
D = 64
DP = 128
NC = 2
NS = 16
NW = NC * NS
BPW = B // NW
L = 16
USER_N = 100000
ITEM_N = 1000000
IT_FULL = ITEM_N // DP
IT_B = IT_FULL + 1
US_FULL = USER_N // DP
US_B = US_FULL + 1
SCR = B + NW
SEG = 4096
NSEG = B // SEG
FB = 64

_i32 = jnp.int32


def _phase1(u_h, i_h, j_h, utabT, itabT, utailT, itailT, su, si, sj,
            chunk, hist_i, off_i, cur_i, hist_j, off_j, cur_j,
            hist_u, off_u, cur_u,
            ent_i, ent_j, ent_u, slab, slabB, fbu, fbi, fbj, du, di, dj,
            sem0, sem1, fsem):
    wid = lax.axis_index("s") * NC + lax.axis_index("c")
    dummy = B + wid
    lane = lax.iota(_i32, L)
    lane0 = lane == 0
    ones = jnp.ones((L,), _i32)

    it_lo = (wid * IT_B) // NW
    it_hi = ((wid + 1) * IT_B) // NW
    us_lo = (wid * US_B) // NW
    us_hi = ((wid + 1) * US_B) // NW

    def zero(ref, n16):
        for t in range(n16):
            ref[pl.ds(t * L, L)] = jnp.zeros((L,), _i32)

    def fill(ref, n16, val):
        for t in range(n16):
            ref[pl.ds(t * L, L)] = jnp.full((L,), val, _i32)

    zero(hist_i, 16)
    zero(hist_j, 16)
    zero(hist_u, 2)
    fill(du, FB // L, dummy)
    fill(di, FB // L, dummy)
    fill(dj, FB // L, dummy)

    UNR = 8

    def scan_hist(src_h, hist, lo, hi):
        for seg in range(NSEG):
            pltpu.sync_copy(src_h.at[pl.ds(seg * SEG, SEG)], chunk)

            def vec(t, carry):
                for s in range(UNR):
                    v = chunk[pl.ds(t * L * UNR + s * L, L)]
                    blk = lax.shift_right_logical(v, 7)
                    m = (blk >= lo) & (blk < hi)
                    plsc.addupdate_scatter(
                        hist, [jnp.where(m, blk - lo, 0)], ones, mask=m)
                return carry

            lax.fori_loop(0, SEG // L // UNR, vec, 0)

    scan_hist(u_h, hist_u, us_lo, us_hi)
    scan_hist(i_h, hist_i, it_lo, it_hi)
    scan_hist(j_h, hist_j, it_lo, it_hi)

    def prefix(hist, off, cur, n16):
        run = _i32(0)
        for t in range(n16):
            v = hist[pl.ds(t * L, L)]
            cs = plsc.cumsum(v)
            ex = cs - v + run
            off[pl.ds(t * L, L)] = ex
            cur[pl.ds(t * L, L)] = ex
            run = run + cs[15]

    prefix(hist_i, off_i, cur_i, 16)
    prefix(hist_j, off_j, cur_j, 16)
    prefix(hist_u, off_u, cur_u, 2)

    rank_base = plsc.scan_count(jnp.zeros((L,), _i32))[0][0]

    def scan_emit(src_h, ent, cur, lo, hi):
        for seg in range(NSEG):
            pltpu.sync_copy(src_h.at[pl.ds(seg * SEG, SEG)], chunk)

            def vec(t, carry):
                for s in range(UNR):
                    b0 = seg * SEG + t * L * UNR + s * L
                    v = chunk[pl.ds(t * L * UNR + s * L, L)]
                    blk = lax.shift_right_logical(v, 7)
                    m = (blk >= lo) & (blk < hi)
                    rank, _ = plsc.scan_count(blk, m)
                    relv = jnp.where(m, blk - lo, 0)
                    base = plsc.load_gather(cur, [relv])
                    slot = base + rank - rank_base
                    entry = ((b0 + lane) << 7) | (v & 127)
                    plsc.store_scatter(ent, [slot], entry, mask=m)
                    plsc.addupdate_scatter(cur, [relv], ones, mask=m)
                return carry

            lax.fori_loop(0, SEG // L // UNR, vec, 0)

    scan_emit(u_h, ent_u, cur_u, us_lo, us_hi)
    scan_emit(i_h, ent_i, cur_i, it_lo, it_hi)
    scan_emit(j_h, ent_j, cur_j, it_lo, it_hi)

    def flush(fbref, dref, tgt):
        pltpu.async_copy(fbref, tgt.at[dref], fsem).wait()
        fill(dref, FB // L, dummy)

    def append(fbref, dref, tgt, cols, b, fc, valid):
        @pl.when(valid)
        def _():
            for c in range(4):
                fbref[fc, pl.ds(c * L, L)] = cols[c]
            plsc.store_scatter(dref, [jnp.full((L,), fc, _i32)],
                               jnp.full((L,), b, _i32), mask=lane0)
        fc = fc + jnp.where(valid, _i32(1), _i32(0))

        def do_flush(_):
            flush(fbref, dref, tgt)
            return _i32(0)

        return lax.cond(fc == FB, do_flush, lambda _: fc, 0)

    def ent_section(slabref, ent, offref, t, fbref, dref, tgt, fc0):
        s = plsc.load_gather(offref, [jnp.full((L,), t, _i32)])[0]
        e = plsc.load_gather(offref, [jnp.full((L,), t + 1, _i32)])[0]

        def grp(g, fc):
            ei = s + g * L
            ev = ent[pl.ds(ei, L)]
            for k in range(L):
                entk = ev[k]
                b = lax.shift_right_logical(entk, 7)
                lnv = jnp.full((L,), entk & 127, _i32)
                cols = [plsc.load_gather(slabref, [c + lane, lnv])
                        for c in (0, 16, 32, 48)]
                fc = append(fbref, dref, tgt, cols, b, fc, (ei + k) < e)
            return fc

        return lax.fori_loop(0, (e - s + L - 1) // L, grp, fc0)

    def issue(tabT, tailT, full_blocks, blk, slabref, semref):
        def cp_tail(_):
            pltpu.async_copy(tailT, slabref, semref)
            return 0

        def cp_full(_):
            pltpu.async_copy(tabT.at[:, pl.ds(blk * DP, DP)], slabref, semref)
            return 0

        lax.cond(blk == full_blocks, cp_tail, cp_full, 0)

    def drain(slabref, semref):
        pltpu.make_async_copy(utabT.at[:, pl.ds(0, DP)], slabref, semref).wait()

    def run_blocks(tabT, tailT, full_blocks, lo, hi, slab0, slab1,
                   sem0, sem1, proc, fcs0):
        nb = hi - lo
        issue(tabT, tailT, full_blocks, lo, slab0, sem0)

        def pairloop(t2, fcs):
            blk0 = lo + 2 * t2
            blk1 = blk0 + 1

            @pl.when(blk1 < hi)
            def _():
                issue(tabT, tailT, full_blocks, blk1, slab1, sem1)

            drain(slab0, sem0)
            fcs = proc(blk0 - lo, slab0, fcs)

            @pl.when(blk0 + 2 < hi)
            def _():
                issue(tabT, tailT, full_blocks, blk0 + 2, slab0, sem0)

            def do1(f):
                drain(slab1, sem1)
                return proc(blk1 - lo, slab1, f)

            return lax.cond(blk1 < hi, do1, lambda f: f, fcs)

        return lax.fori_loop(0, (nb + 1) // 2, pairloop, fcs0)

    def proc_user(t, slabref, fcu):
        return ent_section(slabref, ent_u, off_u, t, fbu, du, su, fcu)

    def proc_item(t, slabref, fcs):
        fci, fcj = fcs
        fci = ent_section(slabref, ent_i, off_i, t, fbi, di, si, fci)
        fcj = ent_section(slabref, ent_j, off_j, t, fbj, dj, sj, fcj)
        return (fci, fcj)

    run_blocks(utabT, utailT, US_FULL, us_lo, us_hi, slab, slabB,
               sem0, sem1, proc_user, _i32(0))
    run_blocks(itabT, itailT, IT_FULL, it_lo, it_hi, slab, slabB,
               sem0, sem1, proc_item, (_i32(0), _i32(0)))

    flush(fbu, du, su)
    flush(fbi, di, si)
    flush(fbj, dj, sj)


def _phase2(su, si, sj, out_i, out_j, ur, vir, vjr, oi, oj, sem):
    wid = lax.axis_index("s") * NC + lax.axis_index("c")
    b0 = wid * BPW
    lane = lax.iota(_i32, L)
    last = lane == (L - 1)

    for k in range(BPW // DP):
        pltpu.sync_copy(su.at[pl.ds(b0 + k * DP, DP)], ur)
        pltpu.sync_copy(si.at[pl.ds(b0 + k * DP, DP)], vir)
        pltpu.sync_copy(sj.at[pl.ds(b0 + k * DP, DP)], vjr)

        def row(r, carry):
            acc_i = jnp.zeros((L,), jnp.float32)
            acc_j = jnp.zeros((L,), jnp.float32)
            for cc in range(D // L):
                u = ur[r, pl.ds(cc * L, L)]
                vi = vir[r, pl.ds(cc * L, L)]
                vj = vjr[r, pl.ds(cc * L, L)]
                acc_i = acc_i + u * vi
                acc_j = acc_j + u * vj
            gidx = jnp.full((L,), k * DP + r, _i32)
            plsc.store_scatter(oi, [gidx], plsc.cumsum(acc_i), mask=last)
            plsc.store_scatter(oj, [gidx], plsc.cumsum(acc_j), mask=last)
            return carry

        lax.fori_loop(0, DP, row, 0)

    pltpu.sync_copy(oi, out_i.at[pl.ds(b0, BPW)])
    pltpu.sync_copy(oj, out_j.at[pl.ds(b0, BPW)])


def kernel(user, item_i, item_j, embed_user_weight, embed_item_weight):
    mesh = plsc.VectorSubcoreMesh(core_axis_name="c", subcore_axis_name="s")
    cp = pltpu.CompilerParams(
        needs_layout_passes=False, use_tc_tiling_on_sc=True)

    run1 = pl.kernel(
        _phase1,
        mesh=mesh,
        compiler_params=cp,
        out_type=(
            jax.ShapeDtypeStruct((SCR, DP), jnp.float32),
            jax.ShapeDtypeStruct((SCR, DP), jnp.float32),
            jax.ShapeDtypeStruct((SCR, DP), jnp.float32),
        ),
        scratch_types=[
            pltpu.VMEM((SEG,), _i32),
            pltpu.VMEM((256,), _i32),
            pltpu.VMEM((256,), _i32),
            pltpu.VMEM((256,), _i32),
            pltpu.VMEM((256,), _i32),
            pltpu.VMEM((256,), _i32),
            pltpu.VMEM((256,), _i32),
            pltpu.VMEM((32,), _i32),
            pltpu.VMEM((32,), _i32),
            pltpu.VMEM((32,), _i32),
            pltpu.VMEM((B + L,), _i32),
            pltpu.VMEM((B + L,), _i32),
            pltpu.VMEM((B + L,), _i32),
            pltpu.VMEM((64, DP), jnp.float32),
            pltpu.VMEM((64, DP), jnp.float32),
            pltpu.VMEM((FB, DP), jnp.float32),
            pltpu.VMEM((FB, DP), jnp.float32),
            pltpu.VMEM((FB, DP), jnp.float32),
            pltpu.VMEM((FB,), _i32),
            pltpu.VMEM((FB,), _i32),
            pltpu.VMEM((FB,), _i32),
            pltpu.SemaphoreType.DMA,
            pltpu.SemaphoreType.DMA,
            pltpu.SemaphoreType.DMA,
        ],
    )

    run2 = pl.kernel(
        _phase2,
        mesh=mesh,
        compiler_params=cp,
        out_type=(
            jax.ShapeDtypeStruct((B,), jnp.float32),
            jax.ShapeDtypeStruct((B,), jnp.float32),
        ),
        scratch_types=[
            pltpu.VMEM((DP, DP), jnp.float32),
            pltpu.VMEM((DP, DP), jnp.float32),
            pltpu.VMEM((DP, DP), jnp.float32),
            pltpu.VMEM((BPW,), jnp.float32),
            pltpu.VMEM((BPW,), jnp.float32),
            pltpu.SemaphoreType.DMA,
        ],
    )

    u = user.astype(_i32)
    i = item_i.astype(_i32)
    j = item_j.astype(_i32)
    utabT = embed_user_weight.T
    itabT = embed_item_weight.T
    utailT = jnp.pad(embed_user_weight[US_FULL * DP:],
                     ((0, DP - (USER_N - US_FULL * DP)), (0, 0))).T
    itailT = jnp.pad(embed_item_weight[IT_FULL * DP:],
                     ((0, DP - (ITEM_N - IT_FULL * DP)), (0, 0))).T
    su, si, sj = run1(u, i, j, utabT, itabT, utailT, itailT)
    return run2(su, si, sj)


# speedup vs baseline: 1.6970x; 1.3646x over previous
import functools

import jax
import jax.numpy as jnp
from jax import lax
from jax.experimental import pallas as pl
from jax.experimental.pallas import tpu as pltpu
from jax.experimental.pallas import tpu_sc as plsc

B = 16384
D = 64
DP = 128
NC = 2
NS = 16
NW = NC * NS
BPW = B // NW
L = 16
USER_N = 100000
ITEM_N = 1000000
IT_FULL = ITEM_N // DP
IT_B = IT_FULL + 1
US_FULL = USER_N // DP
US_B = US_FULL + 1
SCR = B + NW
SEG = 4096
NSEG = B // SEG
FB = 64

_i32 = jnp.int32


def _phase1(u_h, i_h, j_h, utabT, itabT, utailT, itailT, su, si, sj,
            chunk, hist_i, off_i, cur_i, hist_j, off_j, cur_j,
            hist_u, off_u, cur_u,
            ent_i, ent_j, ent_u, slab, slabB, fbu, fbi, fbj, du, di, dj,
            sem0, sem1, fsem):
    wid = lax.axis_index("s") * NC + lax.axis_index("c")
    dummy = B + wid
    lane = lax.iota(_i32, L)
    lane0 = lane == 0
    ones = jnp.ones((L,), _i32)

    it_lo = (wid * IT_B) // NW
    it_hi = ((wid + 1) * IT_B) // NW
    us_lo = (wid * US_B) // NW
    us_hi = ((wid + 1) * US_B) // NW

    def zero(ref, n16):
        for t in range(n16):
            ref[pl.ds(t * L, L)] = jnp.zeros((L,), _i32)

    def fill(ref, n16, val):
        for t in range(n16):
            ref[pl.ds(t * L, L)] = jnp.full((L,), val, _i32)

    zero(hist_i, 16)
    zero(hist_j, 16)
    zero(hist_u, 2)
    fill(du, FB // L, dummy)
    fill(di, FB // L, dummy)
    fill(dj, FB // L, dummy)

    UNR = 8

    def scan_hist(src_h, hist, lo, hi):
        for seg in range(NSEG):
            pltpu.sync_copy(src_h.at[pl.ds(seg * SEG, SEG)], chunk)

            def vec(t, carry):
                for s in range(UNR):
                    v = chunk[pl.ds(t * L * UNR + s * L, L)]
                    blk = lax.shift_right_logical(v, 7)
                    m = (blk >= lo) & (blk < hi)
                    plsc.addupdate_scatter(
                        hist, [jnp.where(m, blk - lo, 0)], ones, mask=m)
                return carry

            lax.fori_loop(0, SEG // L // UNR, vec, 0)

    scan_hist(u_h, hist_u, us_lo, us_hi)
    scan_hist(i_h, hist_i, it_lo, it_hi)
    scan_hist(j_h, hist_j, it_lo, it_hi)

    def prefix(hist, off, cur, n16):
        run = _i32(0)
        for t in range(n16):
            v = hist[pl.ds(t * L, L)]
            cs = plsc.cumsum(v)
            ex = cs - v + run
            off[pl.ds(t * L, L)] = ex
            cur[pl.ds(t * L, L)] = ex
            run = run + cs[15]

    prefix(hist_i, off_i, cur_i, 16)
    prefix(hist_j, off_j, cur_j, 16)
    prefix(hist_u, off_u, cur_u, 2)

    rank_base = plsc.scan_count(jnp.zeros((L,), _i32))[0][0]

    def scan_emit(src_h, ent, cur, lo, hi):
        for seg in range(NSEG):
            pltpu.sync_copy(src_h.at[pl.ds(seg * SEG, SEG)], chunk)

            def vec(t, carry):
                for s in range(UNR):
                    b0 = seg * SEG + t * L * UNR + s * L
                    v = chunk[pl.ds(t * L * UNR + s * L, L)]
                    blk = lax.shift_right_logical(v, 7)
                    m = (blk >= lo) & (blk < hi)
                    rank, _ = plsc.scan_count(blk, m)
                    relv = jnp.where(m, blk - lo, 0)
                    base = plsc.load_gather(cur, [relv])
                    slot = base + rank - rank_base
                    entry = ((b0 + lane) << 7) | (v & 127)
                    plsc.store_scatter(ent, [slot], entry, mask=m)
                    plsc.addupdate_scatter(cur, [relv], ones, mask=m)
                return carry

            lax.fori_loop(0, SEG // L // UNR, vec, 0)

    scan_emit(u_h, ent_u, cur_u, us_lo, us_hi)
    scan_emit(i_h, ent_i, cur_i, it_lo, it_hi)
    scan_emit(j_h, ent_j, cur_j, it_lo, it_hi)

    def flush(fbref, dref, tgt):
        pltpu.async_copy(fbref, tgt.at[dref], fsem).wait()
        fill(dref, FB // L, dummy)

    def append(fbref, dref, tgt, cols, b, fc, valid=None):
        def do_stores():
            for c in range(4):
                fbref[fc, pl.ds(c * L, L)] = cols[c]
            plsc.store_scatter(dref, [jnp.full((L,), fc, _i32)],
                               jnp.full((L,), b, _i32), mask=lane0)
        if valid is None:
            do_stores()
            fc = fc + 1
        else:
            pl.when(valid)(do_stores)
            fc = fc + jnp.where(valid, _i32(1), _i32(0))

        def do_flush(_):
            flush(fbref, dref, tgt)
            return _i32(0)

        return lax.cond(fc == FB, do_flush, lambda _: fc, 0)

    def ent_section(slabref, ent, offref, t, fbref, dref, tgt, fc0):
        s = plsc.load_gather(offref, [jnp.full((L,), t, _i32)])[0]
        e = plsc.load_gather(offref, [jnp.full((L,), t + 1, _i32)])[0]

        def ent_loop(ei, fc):
            entk = ent[pl.ds(ei, L)][0]
            b = lax.shift_right_logical(entk, 7)
            lnv = jnp.full((L,), entk & 127, _i32)
            cols = [plsc.load_gather(slabref, [c + lane, lnv])
                    for c in (0, 16, 32, 48)]
            return append(fbref, dref, tgt, cols, b, fc)

        return lax.fori_loop(s, e, ent_loop, fc0)

    def issue(tabT, tailT, full_blocks, blk, slabref, semref):
        def cp_tail(_):
            pltpu.async_copy(tailT, slabref, semref)
            return 0

        def cp_full(_):
            pltpu.async_copy(tabT.at[:, pl.ds(blk * DP, DP)], slabref, semref)
            return 0

        lax.cond(blk == full_blocks, cp_tail, cp_full, 0)

    def drain(slabref, semref):
        pltpu.make_async_copy(utabT.at[:, pl.ds(0, DP)], slabref, semref).wait()

    def run_blocks(tabT, tailT, full_blocks, lo, hi, slab0, slab1,
                   sem0, sem1, proc, fcs0):
        nb = hi - lo
        issue(tabT, tailT, full_blocks, lo, slab0, sem0)

        def pairloop(t2, fcs):
            blk0 = lo + 2 * t2
            blk1 = blk0 + 1

            @pl.when(blk1 < hi)
            def _():
                issue(tabT, tailT, full_blocks, blk1, slab1, sem1)

            drain(slab0, sem0)
            fcs = proc(blk0 - lo, slab0, fcs)

            @pl.when(blk0 + 2 < hi)
            def _():
                issue(tabT, tailT, full_blocks, blk0 + 2, slab0, sem0)

            def do1(f):
                drain(slab1, sem1)
                return proc(blk1 - lo, slab1, f)

            return lax.cond(blk1 < hi, do1, lambda f: f, fcs)

        return lax.fori_loop(0, (nb + 1) // 2, pairloop, fcs0)

    def proc_user(t, slabref, fcu):
        return ent_section(slabref, ent_u, off_u, t, fbu, du, su, fcu)

    def proc_item(t, slabref, fcs):
        fci, fcj = fcs
        fci = ent_section(slabref, ent_i, off_i, t, fbi, di, si, fci)
        fcj = ent_section(slabref, ent_j, off_j, t, fbj, dj, sj, fcj)
        return (fci, fcj)

    run_blocks(utabT, utailT, US_FULL, us_lo, us_hi, slab, slabB,
               sem0, sem1, proc_user, _i32(0))
    run_blocks(itabT, itailT, IT_FULL, it_lo, it_hi, slab, slabB,
               sem0, sem1, proc_item, (_i32(0), _i32(0)))

    flush(fbu, du, su)
    flush(fbi, di, si)
    flush(fbj, dj, sj)


def _phase2(su, si, sj, out_i, out_j, ur, vir, vjr, oi, oj, sem):
    wid = lax.axis_index("s") * NC + lax.axis_index("c")
    b0 = wid * BPW
    lane = lax.iota(_i32, L)
    last = lane == (L - 1)

    for k in range(BPW // DP):
        pltpu.sync_copy(su.at[pl.ds(b0 + k * DP, DP)], ur)
        pltpu.sync_copy(si.at[pl.ds(b0 + k * DP, DP)], vir)
        pltpu.sync_copy(sj.at[pl.ds(b0 + k * DP, DP)], vjr)

        def row(r, carry):
            acc_i = jnp.zeros((L,), jnp.float32)
            acc_j = jnp.zeros((L,), jnp.float32)
            for cc in range(D // L):
                u = ur[r, pl.ds(cc * L, L)]
                vi = vir[r, pl.ds(cc * L, L)]
                vj = vjr[r, pl.ds(cc * L, L)]
                acc_i = acc_i + u * vi
                acc_j = acc_j + u * vj
            gidx = jnp.full((L,), k * DP + r, _i32)
            plsc.store_scatter(oi, [gidx], plsc.cumsum(acc_i), mask=last)
            plsc.store_scatter(oj, [gidx], plsc.cumsum(acc_j), mask=last)
            return carry

        lax.fori_loop(0, DP, row, 0)

    pltpu.sync_copy(oi, out_i.at[pl.ds(b0, BPW)])
    pltpu.sync_copy(oj, out_j.at[pl.ds(b0, BPW)])


def kernel(user, item_i, item_j, embed_user_weight, embed_item_weight):
    mesh = plsc.VectorSubcoreMesh(core_axis_name="c", subcore_axis_name="s")
    cp = pltpu.CompilerParams(
        needs_layout_passes=False, use_tc_tiling_on_sc=True)

    run1 = pl.kernel(
        _phase1,
        mesh=mesh,
        compiler_params=cp,
        out_type=(
            jax.ShapeDtypeStruct((SCR, DP), jnp.float32),
            jax.ShapeDtypeStruct((SCR, DP), jnp.float32),
            jax.ShapeDtypeStruct((SCR, DP), jnp.float32),
        ),
        scratch_types=[
            pltpu.VMEM((SEG,), _i32),
            pltpu.VMEM((256,), _i32),
            pltpu.VMEM((256,), _i32),
            pltpu.VMEM((256,), _i32),
            pltpu.VMEM((256,), _i32),
            pltpu.VMEM((256,), _i32),
            pltpu.VMEM((256,), _i32),
            pltpu.VMEM((32,), _i32),
            pltpu.VMEM((32,), _i32),
            pltpu.VMEM((32,), _i32),
            pltpu.VMEM((B + L,), _i32),
            pltpu.VMEM((B + L,), _i32),
            pltpu.VMEM((B + L,), _i32),
            pltpu.VMEM((64, DP), jnp.float32),
            pltpu.VMEM((64, DP), jnp.float32),
            pltpu.VMEM((FB, DP), jnp.float32),
            pltpu.VMEM((FB, DP), jnp.float32),
            pltpu.VMEM((FB, DP), jnp.float32),
            pltpu.VMEM((FB,), _i32),
            pltpu.VMEM((FB,), _i32),
            pltpu.VMEM((FB,), _i32),
            pltpu.SemaphoreType.DMA,
            pltpu.SemaphoreType.DMA,
            pltpu.SemaphoreType.DMA,
        ],
    )

    run2 = pl.kernel(
        _phase2,
        mesh=mesh,
        compiler_params=cp,
        out_type=(
            jax.ShapeDtypeStruct((B,), jnp.float32),
            jax.ShapeDtypeStruct((B,), jnp.float32),
        ),
        scratch_types=[
            pltpu.VMEM((DP, DP), jnp.float32),
            pltpu.VMEM((DP, DP), jnp.float32),
            pltpu.VMEM((DP, DP), jnp.float32),
            pltpu.VMEM((BPW,), jnp.float32),
            pltpu.VMEM((BPW,), jnp.float32),
            pltpu.SemaphoreType.DMA,
        ],
    )

    u = user.astype(_i32)
    i = item_i.astype(_i32)
    j = item_j.astype(_i32)
    utabT = embed_user_weight.T
    itabT = embed_item_weight.T
    utailT = jnp.pad(embed_user_weight[US_FULL * DP:],
                     ((0, DP - (USER_N - US_FULL * DP)), (0, 0))).T
    itailT = jnp.pad(embed_item_weight[IT_FULL * DP:],
                     ((0, DP - (ITEM_N - IT_FULL * DP)), (0, 0))).T
    su, si, sj = run1(u, i, j, utabT, itabT, utailT, itailT)
    return run2(su, si, sj)


# speedup vs baseline: 1.8974x; 1.1181x over previous
import functools

import jax
import jax.numpy as jnp
from jax import lax
from jax.experimental import pallas as pl
from jax.experimental.pallas import tpu as pltpu
from jax.experimental.pallas import tpu_sc as plsc

B = 16384
D = 64
DP = 128
NC = 2
NS = 16
NW = NC * NS
BPW = B // NW
L = 16
USER_N = 100000
ITEM_N = 1000000
BW = 256
IT_FULL = ITEM_N // BW
IT_B = IT_FULL + 1
US_FULL = USER_N // BW
US_B = US_FULL + 1
SCR = B + NW
SEG = 4096
NSEG = B // SEG
FB = 64

_i32 = jnp.int32


def _phase1(u_h, i_h, j_h, utabT, itabT, utailT, itailT, su, si, sj,
            chunk, hist_i, off_i, cur_i, hist_j, off_j, cur_j,
            hist_u, off_u, cur_u,
            ent_i, ent_j, ent_u, slab, slabB, fbu, fbi, fbj, du, di, dj,
            sem0, sem1, fsem):
    wid = lax.axis_index("s") * NC + lax.axis_index("c")
    dummy = B + wid
    lane = lax.iota(_i32, L)
    lane0 = lane == 0
    ones = jnp.ones((L,), _i32)

    it_lo = (wid * IT_B) // NW
    it_hi = ((wid + 1) * IT_B) // NW
    us_lo = (wid * US_B) // NW
    us_hi = ((wid + 1) * US_B) // NW

    def zero(ref, n16):
        for t in range(n16):
            ref[pl.ds(t * L, L)] = jnp.zeros((L,), _i32)

    def fill(ref, n16, val):
        for t in range(n16):
            ref[pl.ds(t * L, L)] = jnp.full((L,), val, _i32)

    zero(hist_i, 8)
    zero(hist_j, 8)
    zero(hist_u, 1)
    fill(du, FB // L, dummy)
    fill(di, FB // L, dummy)
    fill(dj, FB // L, dummy)

    UNR = 8

    def scan_hist(src_h, hist, lo, hi):
        for seg in range(NSEG):
            pltpu.sync_copy(src_h.at[pl.ds(seg * SEG, SEG)], chunk)

            def vec(t, carry):
                for s in range(UNR):
                    v = chunk[pl.ds(t * L * UNR + s * L, L)]
                    blk = lax.shift_right_logical(v, 8)
                    m = (blk >= lo) & (blk < hi)
                    plsc.addupdate_scatter(
                        hist, [jnp.where(m, blk - lo, 0)], ones, mask=m)
                return carry

            lax.fori_loop(0, SEG // L // UNR, vec, 0)

    scan_hist(u_h, hist_u, us_lo, us_hi)
    scan_hist(i_h, hist_i, it_lo, it_hi)
    scan_hist(j_h, hist_j, it_lo, it_hi)

    def prefix(hist, off, cur, n16):
        run = _i32(0)
        for t in range(n16):
            v = hist[pl.ds(t * L, L)]
            cs = plsc.cumsum(v)
            ex = cs - v + run
            off[pl.ds(t * L, L)] = ex
            cur[pl.ds(t * L, L)] = ex
            run = run + cs[15]

    prefix(hist_i, off_i, cur_i, 8)
    prefix(hist_j, off_j, cur_j, 8)
    prefix(hist_u, off_u, cur_u, 1)

    rank_base = plsc.scan_count(jnp.zeros((L,), _i32))[0][0]

    def scan_emit(src_h, ent, cur, lo, hi):
        for seg in range(NSEG):
            pltpu.sync_copy(src_h.at[pl.ds(seg * SEG, SEG)], chunk)

            def vec(t, carry):
                for s in range(UNR):
                    b0 = seg * SEG + t * L * UNR + s * L
                    v = chunk[pl.ds(t * L * UNR + s * L, L)]
                    blk = lax.shift_right_logical(v, 8)
                    m = (blk >= lo) & (blk < hi)
                    rank, _ = plsc.scan_count(blk, m)
                    relv = jnp.where(m, blk - lo, 0)
                    base = plsc.load_gather(cur, [relv])
                    slot = base + rank - rank_base
                    entry = ((b0 + lane) << 9) | (v & 255)
                    plsc.store_scatter(ent, [slot], entry, mask=m)
                    plsc.addupdate_scatter(cur, [relv], ones, mask=m)
                return carry

            lax.fori_loop(0, SEG // L // UNR, vec, 0)

    scan_emit(u_h, ent_u, cur_u, us_lo, us_hi)
    scan_emit(i_h, ent_i, cur_i, it_lo, it_hi)
    scan_emit(j_h, ent_j, cur_j, it_lo, it_hi)

    def flush(fbref, dref, tgt):
        pltpu.async_copy(fbref, tgt.at[dref], fsem).wait()
        fill(dref, FB // L, dummy)

    def append(fbref, dref, tgt, cols, b, fc, valid=None):
        def do_stores():
            for c in range(4):
                fbref[fc, pl.ds(c * L, L)] = cols[c]
            plsc.store_scatter(dref, [jnp.full((L,), fc, _i32)],
                               jnp.full((L,), b, _i32), mask=lane0)
        if valid is None:
            do_stores()
            fc = fc + 1
        else:
            pl.when(valid)(do_stores)
            fc = fc + jnp.where(valid, _i32(1), _i32(0))

        def do_flush(_):
            flush(fbref, dref, tgt)
            return _i32(0)

        return lax.cond(fc == FB, do_flush, lambda _: fc, 0)

    def ent_section(slabref, ent, offref, t, fbref, dref, tgt, fc0):
        s = plsc.load_gather(offref, [jnp.full((L,), t, _i32)])[0]
        e = plsc.load_gather(offref, [jnp.full((L,), t + 1, _i32)])[0]

        def ent_loop(ei, fc):
            entk = ent[pl.ds(ei, L)][0]
            b = lax.shift_right_logical(entk, 9)
            lnv = jnp.full((L,), entk & 255, _i32)
            cols = [plsc.load_gather(slabref, [c + lane, lnv])
                    for c in (0, 16, 32, 48)]
            return append(fbref, dref, tgt, cols, b, fc)

        return lax.fori_loop(s, e, ent_loop, fc0)

    def issue(tabT, tailT, full_blocks, blk, slabref, semref):
        def cp_tail(_):
            pltpu.async_copy(tailT, slabref, semref)
            return 0

        def cp_full(_):
            pltpu.async_copy(tabT.at[:, pl.ds(blk * BW, BW)], slabref, semref)
            return 0

        lax.cond(blk == full_blocks, cp_tail, cp_full, 0)

    def drain(slabref, semref):
        pltpu.make_async_copy(utabT.at[:, pl.ds(0, BW)], slabref, semref).wait()

    def run_blocks(tabT, tailT, full_blocks, lo, hi, slab0, slab1,
                   sem0, sem1, proc, fcs0):
        nb = hi - lo
        issue(tabT, tailT, full_blocks, lo, slab0, sem0)

        def pairloop(t2, fcs):
            blk0 = lo + 2 * t2
            blk1 = blk0 + 1

            @pl.when(blk1 < hi)
            def _():
                issue(tabT, tailT, full_blocks, blk1, slab1, sem1)

            drain(slab0, sem0)
            fcs = proc(blk0 - lo, slab0, fcs)

            @pl.when(blk0 + 2 < hi)
            def _():
                issue(tabT, tailT, full_blocks, blk0 + 2, slab0, sem0)

            def do1(f):
                drain(slab1, sem1)
                return proc(blk1 - lo, slab1, f)

            return lax.cond(blk1 < hi, do1, lambda f: f, fcs)

        return lax.fori_loop(0, (nb + 1) // 2, pairloop, fcs0)

    def proc_user(t, slabref, fcu):
        return ent_section(slabref, ent_u, off_u, t, fbu, du, su, fcu)

    def proc_item(t, slabref, fcs):
        fci, fcj = fcs
        fci = ent_section(slabref, ent_i, off_i, t, fbi, di, si, fci)
        fcj = ent_section(slabref, ent_j, off_j, t, fbj, dj, sj, fcj)
        return (fci, fcj)

    run_blocks(utabT, utailT, US_FULL, us_lo, us_hi, slab, slabB,
               sem0, sem1, proc_user, _i32(0))
    run_blocks(itabT, itailT, IT_FULL, it_lo, it_hi, slab, slabB,
               sem0, sem1, proc_item, (_i32(0), _i32(0)))

    flush(fbu, du, su)
    flush(fbi, di, si)
    flush(fbj, dj, sj)


def _phase2(su, si, sj, out_i, out_j, ur, vir, vjr, oi, oj, sem):
    wid = lax.axis_index("s") * NC + lax.axis_index("c")
    b0 = wid * BPW
    lane = lax.iota(_i32, L)
    last = lane == (L - 1)

    for k in range(BPW // DP):
        pltpu.sync_copy(su.at[pl.ds(b0 + k * DP, DP)], ur)
        pltpu.sync_copy(si.at[pl.ds(b0 + k * DP, DP)], vir)
        pltpu.sync_copy(sj.at[pl.ds(b0 + k * DP, DP)], vjr)

        def row(r, carry):
            acc_i = jnp.zeros((L,), jnp.float32)
            acc_j = jnp.zeros((L,), jnp.float32)
            for cc in range(D // L):
                u = ur[r, pl.ds(cc * L, L)]
                vi = vir[r, pl.ds(cc * L, L)]
                vj = vjr[r, pl.ds(cc * L, L)]
                acc_i = acc_i + u * vi
                acc_j = acc_j + u * vj
            gidx = jnp.full((L,), k * DP + r, _i32)
            plsc.store_scatter(oi, [gidx], plsc.cumsum(acc_i), mask=last)
            plsc.store_scatter(oj, [gidx], plsc.cumsum(acc_j), mask=last)
            return carry

        lax.fori_loop(0, DP, row, 0)

    pltpu.sync_copy(oi, out_i.at[pl.ds(b0, BPW)])
    pltpu.sync_copy(oj, out_j.at[pl.ds(b0, BPW)])


def kernel(user, item_i, item_j, embed_user_weight, embed_item_weight):
    mesh = plsc.VectorSubcoreMesh(core_axis_name="c", subcore_axis_name="s")
    cp = pltpu.CompilerParams(
        needs_layout_passes=False, use_tc_tiling_on_sc=True)

    run1 = pl.kernel(
        _phase1,
        mesh=mesh,
        compiler_params=cp,
        out_type=(
            jax.ShapeDtypeStruct((SCR, DP), jnp.float32),
            jax.ShapeDtypeStruct((SCR, DP), jnp.float32),
            jax.ShapeDtypeStruct((SCR, DP), jnp.float32),
        ),
        scratch_types=[
            pltpu.VMEM((SEG,), _i32),
            pltpu.VMEM((128,), _i32),
            pltpu.VMEM((128,), _i32),
            pltpu.VMEM((128,), _i32),
            pltpu.VMEM((128,), _i32),
            pltpu.VMEM((128,), _i32),
            pltpu.VMEM((128,), _i32),
            pltpu.VMEM((16,), _i32),
            pltpu.VMEM((16,), _i32),
            pltpu.VMEM((16,), _i32),
            pltpu.VMEM((B + L,), _i32),
            pltpu.VMEM((B + L,), _i32),
            pltpu.VMEM((B + L,), _i32),
            pltpu.VMEM((64, BW), jnp.float32),
            pltpu.VMEM((64, BW), jnp.float32),
            pltpu.VMEM((FB, DP), jnp.float32),
            pltpu.VMEM((FB, DP), jnp.float32),
            pltpu.VMEM((FB, DP), jnp.float32),
            pltpu.VMEM((FB,), _i32),
            pltpu.VMEM((FB,), _i32),
            pltpu.VMEM((FB,), _i32),
            pltpu.SemaphoreType.DMA,
            pltpu.SemaphoreType.DMA,
            pltpu.SemaphoreType.DMA,
        ],
    )

    run2 = pl.kernel(
        _phase2,
        mesh=mesh,
        compiler_params=cp,
        out_type=(
            jax.ShapeDtypeStruct((B,), jnp.float32),
            jax.ShapeDtypeStruct((B,), jnp.float32),
        ),
        scratch_types=[
            pltpu.VMEM((DP, DP), jnp.float32),
            pltpu.VMEM((DP, DP), jnp.float32),
            pltpu.VMEM((DP, DP), jnp.float32),
            pltpu.VMEM((BPW,), jnp.float32),
            pltpu.VMEM((BPW,), jnp.float32),
            pltpu.SemaphoreType.DMA,
        ],
    )

    u = user.astype(_i32)
    i = item_i.astype(_i32)
    j = item_j.astype(_i32)
    utabT = embed_user_weight.T
    itabT = embed_item_weight.T
    utailT = jnp.pad(embed_user_weight[US_FULL * BW:],
                     ((0, BW - (USER_N - US_FULL * BW)), (0, 0))).T
    itailT = jnp.pad(embed_item_weight[IT_FULL * BW:],
                     ((0, BW - (ITEM_N - IT_FULL * BW)), (0, 0))).T
    su, si, sj = run1(u, i, j, utabT, itabT, utailT, itailT)
    return run2(su, si, sj)


# speedup vs baseline: 1.9143x; 1.0089x over previous
import functools

import jax
import jax.numpy as jnp
from jax import lax
from jax.experimental import pallas as pl
from jax.experimental.pallas import tpu as pltpu
from jax.experimental.pallas import tpu_sc as plsc

B = 16384
D = 64
DP = 128
NC = 2
NS = 16
NW = NC * NS
BPW = B // NW
L = 16
USER_N = 100000
ITEM_N = 1000000
BW = 256
IT_FULL = ITEM_N // BW
IT_B = IT_FULL + 1
US_FULL = USER_N // BW
US_B = US_FULL + 1
SCR = B + NW
SEG = 4096
NSEG = B // SEG
FB = 64

_i32 = jnp.int32


def _phase1(u_h, i_h, j_h, utabT, itabT, utailT, itailT, su, si, sj,
            chunk, hist_i, off_i, cur_i, hist_j, off_j, cur_j,
            hist_u, off_u, cur_u,
            ent_i, ent_j, ent_u, slab, slabB, fbu, fbi, fbj, du, di, dj,
            sem0, sem1, fsem):
    wid = lax.axis_index("s") * NC + lax.axis_index("c")
    dummy = B + wid
    lane = lax.iota(_i32, L)
    lane0 = lane == 0
    ones = jnp.ones((L,), _i32)

    it_lo = (wid * IT_B) // NW
    it_hi = ((wid + 1) * IT_B) // NW
    us_lo = (wid * US_B) // NW
    us_hi = ((wid + 1) * US_B) // NW

    def zero(ref, n16):
        for t in range(n16):
            ref[pl.ds(t * L, L)] = jnp.zeros((L,), _i32)

    def fill(ref, n16, val):
        for t in range(n16):
            ref[pl.ds(t * L, L)] = jnp.full((L,), val, _i32)

    zero(hist_i, 8)
    zero(hist_j, 8)
    zero(hist_u, 1)
    fill(du, FB // L, dummy)
    fill(di, FB // L, dummy)
    fill(dj, FB // L, dummy)

    UNR = 8

    def scan_hist(src_h, hist, lo, hi):
        for seg in range(NSEG):
            pltpu.sync_copy(src_h.at[pl.ds(seg * SEG, SEG)], chunk)

            def vec(t, carry):
                for s in range(UNR):
                    v = chunk[pl.ds(t * L * UNR + s * L, L)]
                    blk = lax.shift_right_logical(v, 8)
                    m = (blk >= lo) & (blk < hi)
                    plsc.addupdate_scatter(
                        hist, [jnp.where(m, blk - lo, 0)], ones, mask=m)
                return carry

            lax.fori_loop(0, SEG // L // UNR, vec, 0)

    scan_hist(u_h, hist_u, us_lo, us_hi)
    scan_hist(i_h, hist_i, it_lo, it_hi)
    scan_hist(j_h, hist_j, it_lo, it_hi)

    def prefix(hist, off, cur, n16):
        run = _i32(0)
        for t in range(n16):
            v = hist[pl.ds(t * L, L)]
            cs = plsc.cumsum(v)
            ex = cs - v + run
            off[pl.ds(t * L, L)] = ex
            cur[pl.ds(t * L, L)] = ex
            run = run + cs[15]

    prefix(hist_i, off_i, cur_i, 8)
    prefix(hist_j, off_j, cur_j, 8)
    prefix(hist_u, off_u, cur_u, 1)

    rank_base = plsc.scan_count(jnp.zeros((L,), _i32))[0][0]

    def scan_emit(src_h, ent, cur, lo, hi):
        for seg in range(NSEG):
            pltpu.sync_copy(src_h.at[pl.ds(seg * SEG, SEG)], chunk)

            def vec(t, carry):
                for s in range(UNR):
                    b0 = seg * SEG + t * L * UNR + s * L
                    v = chunk[pl.ds(t * L * UNR + s * L, L)]
                    blk = lax.shift_right_logical(v, 8)
                    m = (blk >= lo) & (blk < hi)
                    rank, _ = plsc.scan_count(blk, m)
                    relv = jnp.where(m, blk - lo, 0)
                    base = plsc.load_gather(cur, [relv])
                    slot = base + rank - rank_base
                    entry = ((b0 + lane) << 9) | (v & 255)
                    plsc.store_scatter(ent, [slot], entry, mask=m)
                    plsc.addupdate_scatter(cur, [relv], ones, mask=m)
                return carry

            lax.fori_loop(0, SEG // L // UNR, vec, 0)

    scan_emit(u_h, ent_u, cur_u, us_lo, us_hi)
    scan_emit(i_h, ent_i, cur_i, it_lo, it_hi)
    scan_emit(j_h, ent_j, cur_j, it_lo, it_hi)

    def flush(fbref, dref, tgt):
        pltpu.async_copy(fbref, tgt.at[dref], fsem).wait()
        fill(dref, FB // L, dummy)

    def append(fbref, dref, tgt, cols, b, fc, valid=None):
        def do_stores():
            for c in range(4):
                fbref[fc, pl.ds(c * L, L)] = cols[c]
            plsc.store_scatter(dref, [jnp.full((L,), fc, _i32)],
                               jnp.full((L,), b, _i32), mask=lane0)
        if valid is None:
            do_stores()
            fc = fc + 1
        else:
            pl.when(valid)(do_stores)
            fc = fc + jnp.where(valid, _i32(1), _i32(0))

        def do_flush(_):
            flush(fbref, dref, tgt)
            return _i32(0)

        return lax.cond(fc == FB, do_flush, lambda _: fc, 0)

    def ent_section(slabref, ent, offref, t, fbref, dref, tgt, fc0):
        se = plsc.load_gather(offref, [t + (lane & 1)])
        s, e = se[0], se[1]

        def ent_loop(g, fc):
            ei = s + g * 2
            ev = ent[pl.ds(ei, L)]
            for k in range(2):
                entk = ev[k]
                b = lax.shift_right_logical(entk, 9)
                lnv = jnp.full((L,), entk & 255, _i32)
                cols = [plsc.load_gather(slabref, [c + lane, lnv])
                        for c in (0, 16, 32, 48)]
                fc = append(fbref, dref, tgt, cols, b, fc,
                            valid=(ei + k) < e)
            return fc

        return lax.fori_loop(0, (e - s + 1) // 2, ent_loop, fc0)

    def issue(tabT, tailT, full_blocks, blk, slabref, semref):
        def cp_tail(_):
            pltpu.async_copy(tailT, slabref, semref)
            return 0

        def cp_full(_):
            pltpu.async_copy(tabT.at[:, pl.ds(blk * BW, BW)], slabref, semref)
            return 0

        lax.cond(blk == full_blocks, cp_tail, cp_full, 0)

    def drain(slabref, semref):
        pltpu.make_async_copy(utabT.at[:, pl.ds(0, BW)], slabref, semref).wait()

    def run_blocks(tabT, tailT, full_blocks, lo, hi, slab0, slab1,
                   sem0, sem1, proc, fcs0):
        nb = hi - lo
        issue(tabT, tailT, full_blocks, lo, slab0, sem0)

        def pairloop(t2, fcs):
            blk0 = lo + 2 * t2
            blk1 = blk0 + 1

            @pl.when(blk1 < hi)
            def _():
                issue(tabT, tailT, full_blocks, blk1, slab1, sem1)

            drain(slab0, sem0)
            fcs = proc(blk0 - lo, slab0, fcs)

            @pl.when(blk0 + 2 < hi)
            def _():
                issue(tabT, tailT, full_blocks, blk0 + 2, slab0, sem0)

            def do1(f):
                drain(slab1, sem1)
                return proc(blk1 - lo, slab1, f)

            return lax.cond(blk1 < hi, do1, lambda f: f, fcs)

        return lax.fori_loop(0, (nb + 1) // 2, pairloop, fcs0)

    def proc_user(t, slabref, fcu):
        return ent_section(slabref, ent_u, off_u, t, fbu, du, su, fcu)

    def proc_item(t, slabref, fcs):
        fci, fcj = fcs
        fci = ent_section(slabref, ent_i, off_i, t, fbi, di, si, fci)
        fcj = ent_section(slabref, ent_j, off_j, t, fbj, dj, sj, fcj)
        return (fci, fcj)

    run_blocks(utabT, utailT, US_FULL, us_lo, us_hi, slab, slabB,
               sem0, sem1, proc_user, _i32(0))
    run_blocks(itabT, itailT, IT_FULL, it_lo, it_hi, slab, slabB,
               sem0, sem1, proc_item, (_i32(0), _i32(0)))

    flush(fbu, du, su)
    flush(fbi, di, si)
    flush(fbj, dj, sj)


def _phase2(su, si, sj, out_i, out_j, ur, vir, vjr, oi, oj, sem):
    wid = lax.axis_index("s") * NC + lax.axis_index("c")
    b0 = wid * BPW
    lane = lax.iota(_i32, L)
    last = lane == (L - 1)

    for k in range(BPW // DP):
        pltpu.sync_copy(su.at[pl.ds(b0 + k * DP, DP)], ur)
        pltpu.sync_copy(si.at[pl.ds(b0 + k * DP, DP)], vir)
        pltpu.sync_copy(sj.at[pl.ds(b0 + k * DP, DP)], vjr)

        def row(r, carry):
            acc_i = jnp.zeros((L,), jnp.float32)
            acc_j = jnp.zeros((L,), jnp.float32)
            for cc in range(D // L):
                u = ur[r, pl.ds(cc * L, L)]
                vi = vir[r, pl.ds(cc * L, L)]
                vj = vjr[r, pl.ds(cc * L, L)]
                acc_i = acc_i + u * vi
                acc_j = acc_j + u * vj
            gidx = jnp.full((L,), k * DP + r, _i32)
            plsc.store_scatter(oi, [gidx], plsc.cumsum(acc_i), mask=last)
            plsc.store_scatter(oj, [gidx], plsc.cumsum(acc_j), mask=last)
            return carry

        lax.fori_loop(0, DP, row, 0)

    pltpu.sync_copy(oi, out_i.at[pl.ds(b0, BPW)])
    pltpu.sync_copy(oj, out_j.at[pl.ds(b0, BPW)])


def kernel(user, item_i, item_j, embed_user_weight, embed_item_weight):
    mesh = plsc.VectorSubcoreMesh(core_axis_name="c", subcore_axis_name="s")
    cp = pltpu.CompilerParams(
        needs_layout_passes=False, use_tc_tiling_on_sc=True)

    run1 = pl.kernel(
        _phase1,
        mesh=mesh,
        compiler_params=cp,
        out_type=(
            jax.ShapeDtypeStruct((SCR, DP), jnp.float32),
            jax.ShapeDtypeStruct((SCR, DP), jnp.float32),
            jax.ShapeDtypeStruct((SCR, DP), jnp.float32),
        ),
        scratch_types=[
            pltpu.VMEM((SEG,), _i32),
            pltpu.VMEM((128,), _i32),
            pltpu.VMEM((128,), _i32),
            pltpu.VMEM((128,), _i32),
            pltpu.VMEM((128,), _i32),
            pltpu.VMEM((128,), _i32),
            pltpu.VMEM((128,), _i32),
            pltpu.VMEM((16,), _i32),
            pltpu.VMEM((16,), _i32),
            pltpu.VMEM((16,), _i32),
            pltpu.VMEM((B + L,), _i32),
            pltpu.VMEM((B + L,), _i32),
            pltpu.VMEM((B + L,), _i32),
            pltpu.VMEM((64, BW), jnp.float32),
            pltpu.VMEM((64, BW), jnp.float32),
            pltpu.VMEM((FB, DP), jnp.float32),
            pltpu.VMEM((FB, DP), jnp.float32),
            pltpu.VMEM((FB, DP), jnp.float32),
            pltpu.VMEM((FB,), _i32),
            pltpu.VMEM((FB,), _i32),
            pltpu.VMEM((FB,), _i32),
            pltpu.SemaphoreType.DMA,
            pltpu.SemaphoreType.DMA,
            pltpu.SemaphoreType.DMA,
        ],
    )

    run2 = pl.kernel(
        _phase2,
        mesh=mesh,
        compiler_params=cp,
        out_type=(
            jax.ShapeDtypeStruct((B,), jnp.float32),
            jax.ShapeDtypeStruct((B,), jnp.float32),
        ),
        scratch_types=[
            pltpu.VMEM((DP, DP), jnp.float32),
            pltpu.VMEM((DP, DP), jnp.float32),
            pltpu.VMEM((DP, DP), jnp.float32),
            pltpu.VMEM((BPW,), jnp.float32),
            pltpu.VMEM((BPW,), jnp.float32),
            pltpu.SemaphoreType.DMA,
        ],
    )

    u = user.astype(_i32)
    i = item_i.astype(_i32)
    j = item_j.astype(_i32)
    utabT = embed_user_weight.T
    itabT = embed_item_weight.T
    utailT = jnp.pad(embed_user_weight[US_FULL * BW:],
                     ((0, BW - (USER_N - US_FULL * BW)), (0, 0))).T
    itailT = jnp.pad(embed_item_weight[IT_FULL * BW:],
                     ((0, BW - (ITEM_N - IT_FULL * BW)), (0, 0))).T
    su, si, sj = run1(u, i, j, utabT, itabT, utailT, itailT)
    return run2(su, si, sj)


# speedup vs baseline: 1.9521x; 1.0197x over previous
import functools

import jax
import jax.numpy as jnp
from jax import lax
from jax.experimental import pallas as pl
from jax.experimental.pallas import tpu as pltpu
from jax.experimental.pallas import tpu_sc as plsc

B = 16384
D = 64
DP = 128
NC = 2
NS = 16
NW = NC * NS
BPW = B // NW
L = 16
USER_N = 100000
ITEM_N = 1000000
BW = 256
IT_FULL = ITEM_N // BW
IT_B = IT_FULL + 1
US_FULL = USER_N // BW
US_B = US_FULL + 1
SCR = B + NW
SEG = 8192
NSEG = B // SEG
FB = 96

_i32 = jnp.int32


def _phase1(u_h, i_h, j_h, utabT, itabT, utailT, itailT, su, si, sj,
            chunk, hist_i, off_i, cur_i, hist_j, off_j, cur_j,
            hist_u, off_u, cur_u,
            ent_i, ent_j, ent_u, slab, slabB, fbu, fbi, fbj, du, di, dj,
            sem0, sem1, fsem):
    wid = lax.axis_index("s") * NC + lax.axis_index("c")
    dummy = B + wid
    lane = lax.iota(_i32, L)
    lane0 = lane == 0
    ones = jnp.ones((L,), _i32)

    it_lo = (wid * IT_B) // NW
    it_hi = ((wid + 1) * IT_B) // NW
    us_lo = (wid * US_B) // NW
    us_hi = ((wid + 1) * US_B) // NW

    def zero(ref, n16):
        for t in range(n16):
            ref[pl.ds(t * L, L)] = jnp.zeros((L,), _i32)

    def fill(ref, n16, val):
        for t in range(n16):
            ref[pl.ds(t * L, L)] = jnp.full((L,), val, _i32)

    zero(hist_i, 8)
    zero(hist_j, 8)
    zero(hist_u, 1)
    fill(du, FB // L, dummy)
    fill(di, FB // L, dummy)
    fill(dj, FB // L, dummy)

    UNR = 16

    def scan_hist(src_h, hist, lo, hi):
        for seg in range(NSEG):
            pltpu.sync_copy(src_h.at[pl.ds(seg * SEG, SEG)], chunk)

            def vec(t, carry):
                for s in range(UNR):
                    v = chunk[pl.ds(t * L * UNR + s * L, L)]
                    blk = lax.shift_right_logical(v, 8)
                    m = (blk >= lo) & (blk < hi)
                    plsc.addupdate_scatter(
                        hist, [jnp.where(m, blk - lo, 0)], ones, mask=m)
                return carry

            lax.fori_loop(0, SEG // L // UNR, vec, 0)

    scan_hist(u_h, hist_u, us_lo, us_hi)
    scan_hist(i_h, hist_i, it_lo, it_hi)
    scan_hist(j_h, hist_j, it_lo, it_hi)

    def prefix(hist, off, cur, n16):
        run = _i32(0)
        for t in range(n16):
            v = hist[pl.ds(t * L, L)]
            cs = plsc.cumsum(v)
            ex = cs - v + run
            off[pl.ds(t * L, L)] = ex
            cur[pl.ds(t * L, L)] = ex
            run = run + cs[15]

    prefix(hist_i, off_i, cur_i, 8)
    prefix(hist_j, off_j, cur_j, 8)
    prefix(hist_u, off_u, cur_u, 1)

    rank_base = plsc.scan_count(jnp.zeros((L,), _i32))[0][0]

    def scan_emit(src_h, ent, cur, lo, hi):
        for seg in range(NSEG):
            pltpu.sync_copy(src_h.at[pl.ds(seg * SEG, SEG)], chunk)

            def vec(t, carry):
                for s in range(UNR):
                    b0 = seg * SEG + t * L * UNR + s * L
                    v = chunk[pl.ds(t * L * UNR + s * L, L)]
                    blk = lax.shift_right_logical(v, 8)
                    m = (blk >= lo) & (blk < hi)
                    rank, _ = plsc.scan_count(blk, m)
                    relv = jnp.where(m, blk - lo, 0)
                    base = plsc.load_gather(cur, [relv])
                    slot = base + rank - rank_base
                    entry = ((b0 + lane) << 9) | (v & 255)
                    plsc.store_scatter(ent, [slot], entry, mask=m)
                    plsc.addupdate_scatter(cur, [relv], ones, mask=m)
                return carry

            lax.fori_loop(0, SEG // L // UNR, vec, 0)

    scan_emit(u_h, ent_u, cur_u, us_lo, us_hi)
    scan_emit(i_h, ent_i, cur_i, it_lo, it_hi)
    scan_emit(j_h, ent_j, cur_j, it_lo, it_hi)

    def flush(fbref, dref, tgt):
        pltpu.async_copy(fbref, tgt.at[dref], fsem).wait()
        fill(dref, FB // L, dummy)

    def append(fbref, dref, tgt, cols, b, fc, valid=None):
        def do_stores():
            for c in range(4):
                fbref[fc, pl.ds(c * L, L)] = cols[c]
            plsc.store_scatter(dref, [jnp.full((L,), fc, _i32)],
                               jnp.full((L,), b, _i32), mask=lane0)
        if valid is None:
            do_stores()
            fc = fc + 1
        else:
            pl.when(valid)(do_stores)
            fc = fc + jnp.where(valid, _i32(1), _i32(0))

        def do_flush(_):
            flush(fbref, dref, tgt)
            return _i32(0)

        return lax.cond(fc == FB, do_flush, lambda _: fc, 0)

    def ent_section(slabref, ent, offref, t, fbref, dref, tgt, fc0):
        se = plsc.load_gather(offref, [t + (lane & 1)])
        s, e = se[0], se[1]

        def ent_loop(g, fc):
            ei = s + g * 2
            ev = ent[pl.ds(ei, L)]
            for k in range(2):
                entk = ev[k]
                b = lax.shift_right_logical(entk, 9)
                lnv = jnp.full((L,), entk & 255, _i32)
                cols = [plsc.load_gather(slabref, [c + lane, lnv])
                        for c in (0, 16, 32, 48)]
                fc = append(fbref, dref, tgt, cols, b, fc,
                            valid=(ei + k) < e)
            return fc

        return lax.fori_loop(0, (e - s + 1) // 2, ent_loop, fc0)

    def issue(tabT, tailT, full_blocks, blk, slabref, semref):
        def cp_tail(_):
            pltpu.async_copy(tailT, slabref, semref)
            return 0

        def cp_full(_):
            pltpu.async_copy(tabT.at[:, pl.ds(blk * BW, BW)], slabref, semref)
            return 0

        lax.cond(blk == full_blocks, cp_tail, cp_full, 0)

    def drain(slabref, semref):
        pltpu.make_async_copy(utabT.at[:, pl.ds(0, BW)], slabref, semref).wait()

    def run_blocks(tabT, tailT, full_blocks, lo, hi, slab0, slab1,
                   sem0, sem1, proc, fcs0):
        nb = hi - lo
        issue(tabT, tailT, full_blocks, lo, slab0, sem0)

        def pairloop(t2, fcs):
            blk0 = lo + 2 * t2
            blk1 = blk0 + 1

            @pl.when(blk1 < hi)
            def _():
                issue(tabT, tailT, full_blocks, blk1, slab1, sem1)

            drain(slab0, sem0)
            fcs = proc(blk0 - lo, slab0, fcs)

            @pl.when(blk0 + 2 < hi)
            def _():
                issue(tabT, tailT, full_blocks, blk0 + 2, slab0, sem0)

            def do1(f):
                drain(slab1, sem1)
                return proc(blk1 - lo, slab1, f)

            return lax.cond(blk1 < hi, do1, lambda f: f, fcs)

        return lax.fori_loop(0, (nb + 1) // 2, pairloop, fcs0)

    def proc_user(t, slabref, fcu):
        return ent_section(slabref, ent_u, off_u, t, fbu, du, su, fcu)

    def proc_item(t, slabref, fcs):
        fci, fcj = fcs
        fci = ent_section(slabref, ent_i, off_i, t, fbi, di, si, fci)
        fcj = ent_section(slabref, ent_j, off_j, t, fbj, dj, sj, fcj)
        return (fci, fcj)

    run_blocks(utabT, utailT, US_FULL, us_lo, us_hi, slab, slabB,
               sem0, sem1, proc_user, _i32(0))
    run_blocks(itabT, itailT, IT_FULL, it_lo, it_hi, slab, slabB,
               sem0, sem1, proc_item, (_i32(0), _i32(0)))

    flush(fbu, du, su)
    flush(fbi, di, si)
    flush(fbj, dj, sj)


def _phase2(su, si, sj, out_i, out_j, ur, vir, vjr, oi, oj, sem):
    wid = lax.axis_index("s") * NC + lax.axis_index("c")
    b0 = wid * BPW
    lane = lax.iota(_i32, L)
    last = lane == (L - 1)

    for k in range(BPW // DP):
        pltpu.sync_copy(su.at[pl.ds(b0 + k * DP, DP)], ur)
        pltpu.sync_copy(si.at[pl.ds(b0 + k * DP, DP)], vir)
        pltpu.sync_copy(sj.at[pl.ds(b0 + k * DP, DP)], vjr)

        def row(r, carry):
            acc_i = jnp.zeros((L,), jnp.float32)
            acc_j = jnp.zeros((L,), jnp.float32)
            for cc in range(D // L):
                u = ur[r, pl.ds(cc * L, L)]
                vi = vir[r, pl.ds(cc * L, L)]
                vj = vjr[r, pl.ds(cc * L, L)]
                acc_i = acc_i + u * vi
                acc_j = acc_j + u * vj
            gidx = jnp.full((L,), k * DP + r, _i32)
            plsc.store_scatter(oi, [gidx], plsc.cumsum(acc_i), mask=last)
            plsc.store_scatter(oj, [gidx], plsc.cumsum(acc_j), mask=last)
            return carry

        lax.fori_loop(0, DP, row, 0)

    pltpu.sync_copy(oi, out_i.at[pl.ds(b0, BPW)])
    pltpu.sync_copy(oj, out_j.at[pl.ds(b0, BPW)])


def kernel(user, item_i, item_j, embed_user_weight, embed_item_weight):
    mesh = plsc.VectorSubcoreMesh(core_axis_name="c", subcore_axis_name="s")
    cp = pltpu.CompilerParams(
        needs_layout_passes=False, use_tc_tiling_on_sc=True)

    run1 = pl.kernel(
        _phase1,
        mesh=mesh,
        compiler_params=cp,
        out_type=(
            jax.ShapeDtypeStruct((SCR, DP), jnp.float32),
            jax.ShapeDtypeStruct((SCR, DP), jnp.float32),
            jax.ShapeDtypeStruct((SCR, DP), jnp.float32),
        ),
        scratch_types=[
            pltpu.VMEM((SEG,), _i32),
            pltpu.VMEM((128,), _i32),
            pltpu.VMEM((128,), _i32),
            pltpu.VMEM((128,), _i32),
            pltpu.VMEM((128,), _i32),
            pltpu.VMEM((128,), _i32),
            pltpu.VMEM((128,), _i32),
            pltpu.VMEM((16,), _i32),
            pltpu.VMEM((16,), _i32),
            pltpu.VMEM((16,), _i32),
            pltpu.VMEM((B + L,), _i32),
            pltpu.VMEM((B + L,), _i32),
            pltpu.VMEM((B + L,), _i32),
            pltpu.VMEM((64, BW), jnp.float32),
            pltpu.VMEM((64, BW), jnp.float32),
            pltpu.VMEM((FB, DP), jnp.float32),
            pltpu.VMEM((FB, DP), jnp.float32),
            pltpu.VMEM((FB, DP), jnp.float32),
            pltpu.VMEM((FB,), _i32),
            pltpu.VMEM((FB,), _i32),
            pltpu.VMEM((FB,), _i32),
            pltpu.SemaphoreType.DMA,
            pltpu.SemaphoreType.DMA,
            pltpu.SemaphoreType.DMA,
        ],
    )

    run2 = pl.kernel(
        _phase2,
        mesh=mesh,
        compiler_params=cp,
        out_type=(
            jax.ShapeDtypeStruct((B,), jnp.float32),
            jax.ShapeDtypeStruct((B,), jnp.float32),
        ),
        scratch_types=[
            pltpu.VMEM((DP, DP), jnp.float32),
            pltpu.VMEM((DP, DP), jnp.float32),
            pltpu.VMEM((DP, DP), jnp.float32),
            pltpu.VMEM((BPW,), jnp.float32),
            pltpu.VMEM((BPW,), jnp.float32),
            pltpu.SemaphoreType.DMA,
        ],
    )

    u = user.astype(_i32)
    i = item_i.astype(_i32)
    j = item_j.astype(_i32)
    utabT = embed_user_weight.T
    itabT = embed_item_weight.T
    utailT = jnp.pad(embed_user_weight[US_FULL * BW:],
                     ((0, BW - (USER_N - US_FULL * BW)), (0, 0))).T
    itailT = jnp.pad(embed_item_weight[IT_FULL * BW:],
                     ((0, BW - (ITEM_N - IT_FULL * BW)), (0, 0))).T
    su, si, sj = run1(u, i, j, utabT, itabT, utailT, itailT)
    return run2(su, si, sj)


# speedup vs baseline: 2.0194x; 1.0345x over previous
import functools

import jax
import jax.numpy as jnp
from jax import lax
from jax.experimental import pallas as pl
from jax.experimental.pallas import tpu as pltpu
from jax.experimental.pallas import tpu_sc as plsc

B = 16384
D = 64
DP = 128
NC = 2
NS = 16
NW = NC * NS
BPW = B // NW
L = 16
USER_N = 100000
ITEM_N = 1000000
BW = 256
IT_FULL = ITEM_N // BW
IT_B = IT_FULL + 1
US_FULL = USER_N // BW
US_B = US_FULL + 1
SCR = B + NW
SEG = 8192
NSEG = B // SEG
FB = 96

_i32 = jnp.int32


def _phase1(u_h, i_h, j_h, utabT, itabT, utailT, itailT, su, si, sj,
            chunk, hist_i, off_i, cur_i, hist_j, off_j, cur_j,
            hist_u, off_u, cur_u,
            ent_i, ent_j, ent_u, slab, slabB, fbu, fbi, fbj, du, di, dj,
            sem0, sem1, fsem):
    wid = lax.axis_index("s") * NC + lax.axis_index("c")
    dummy = B + wid
    lane = lax.iota(_i32, L)
    lane0 = lane == 0
    ones = jnp.ones((L,), _i32)

    it_lo = (wid * IT_B) // NW
    it_hi = ((wid + 1) * IT_B) // NW
    us_lo = (wid * US_B) // NW
    us_hi = ((wid + 1) * US_B) // NW

    def zero(ref, n16):
        for t in range(n16):
            ref[pl.ds(t * L, L)] = jnp.zeros((L,), _i32)

    def fill(ref, n16, val):
        for t in range(n16):
            ref[pl.ds(t * L, L)] = jnp.full((L,), val, _i32)

    zero(hist_i, 8)
    zero(hist_j, 8)
    zero(hist_u, 1)
    fill(du, FB // L, dummy)
    fill(di, FB // L, dummy)
    fill(dj, FB // L, dummy)

    UNR = 16

    def scan_hist(src_h, hist, lo, hi):
        for seg in range(NSEG):
            pltpu.sync_copy(src_h.at[pl.ds(seg * SEG, SEG)], chunk)

            def vec(t, carry):
                for s in range(UNR):
                    v = chunk[pl.ds(t * L * UNR + s * L, L)]
                    blk = lax.shift_right_logical(v, 8)
                    m = (blk >= lo) & (blk < hi)
                    plsc.addupdate_scatter(
                        hist, [jnp.where(m, blk - lo, 0)], ones, mask=m)
                return carry

            lax.fori_loop(0, SEG // L // UNR, vec, 0)

    scan_hist(u_h, hist_u, us_lo, us_hi)
    scan_hist(i_h, hist_i, it_lo, it_hi)
    scan_hist(j_h, hist_j, it_lo, it_hi)

    def prefix(hist, off, cur, n16):
        run = _i32(0)
        for t in range(n16):
            v = hist[pl.ds(t * L, L)]
            cs = plsc.cumsum(v)
            ex = cs - v + run
            off[pl.ds(t * L, L)] = ex
            cur[pl.ds(t * L, L)] = ex
            run = run + cs[15]

    prefix(hist_i, off_i, cur_i, 8)
    prefix(hist_j, off_j, cur_j, 8)
    prefix(hist_u, off_u, cur_u, 1)

    rank_base = plsc.scan_count(jnp.zeros((L,), _i32))[0][0]

    def scan_emit(src_h, ent, cur, lo, hi):
        for seg in range(NSEG):
            pltpu.sync_copy(src_h.at[pl.ds(seg * SEG, SEG)], chunk)

            def vec(t, carry):
                for s in range(UNR):
                    b0 = seg * SEG + t * L * UNR + s * L
                    v = chunk[pl.ds(t * L * UNR + s * L, L)]
                    blk = lax.shift_right_logical(v, 8)
                    m = (blk >= lo) & (blk < hi)
                    rank, _ = plsc.scan_count(blk, m)
                    relv = jnp.where(m, blk - lo, 0)
                    base = plsc.load_gather(cur, [relv])
                    slot = base + rank - rank_base
                    entry = ((b0 + lane) << 9) | (v & 255)
                    plsc.store_scatter(ent, [slot], entry, mask=m)
                    plsc.addupdate_scatter(cur, [relv], ones, mask=m)
                return carry

            lax.fori_loop(0, SEG // L // UNR, vec, 0)

    scan_emit(u_h, ent_u, cur_u, us_lo, us_hi)
    scan_emit(i_h, ent_i, cur_i, it_lo, it_hi)
    scan_emit(j_h, ent_j, cur_j, it_lo, it_hi)

    def flush(fbref, dref, tgt):
        pltpu.async_copy(fbref, tgt.at[dref], fsem).wait()
        fill(dref, FB // L, dummy)

    def append(fbref, dref, tgt, cols, b, fc, valid=None):
        def do_stores():
            for c in range(4):
                fbref[fc, pl.ds(c * L, L)] = cols[c]
            plsc.store_scatter(dref, [jnp.full((L,), fc, _i32)],
                               jnp.full((L,), b, _i32), mask=lane0)
        if valid is None:
            do_stores()
            fc = fc + 1
        else:
            pl.when(valid)(do_stores)
            fc = fc + jnp.where(valid, _i32(1), _i32(0))

        def do_flush(_):
            flush(fbref, dref, tgt)
            return _i32(0)

        return lax.cond(fc == FB, do_flush, lambda _: fc, 0)

    def ent_section(slabref, ent, offref, t, fbref, dref, tgt, fc0):
        se = plsc.load_gather(offref, [t + (lane & 1)])
        s, e = se[0], se[1]

        def ent_loop(g, fc):
            ei = s + g * 2
            ev = ent[pl.ds(ei, L)]
            for k in range(2):
                entk = ev[k]
                b = lax.shift_right_logical(entk, 9)
                lnv = jnp.full((L,), entk & 255, _i32)
                cols = [plsc.load_gather(slabref, [c + lane, lnv])
                        for c in (0, 16, 32, 48)]
                fc = append(fbref, dref, tgt, cols, b, fc,
                            valid=(ei + k) < e)
            return fc

        return lax.fori_loop(0, (e - s + 1) // 2, ent_loop, fc0)

    def issue(tabT, tailT, full_blocks, blk, slabref, semref):
        def cp_tail(_):
            pltpu.async_copy(tailT, slabref, semref)
            return 0

        def cp_full(_):
            pltpu.async_copy(tabT.at[:, pl.ds(blk * BW, BW)], slabref, semref)
            return 0

        lax.cond(blk == full_blocks, cp_tail, cp_full, 0)

    def drain(slabref, semref):
        pltpu.make_async_copy(utabT.at[:, pl.ds(0, BW)], slabref, semref).wait()

    def run_blocks(tabT, tailT, full_blocks, lo, hi, slab0, slab1,
                   sem0, sem1, proc, fcs0):
        nb = hi - lo
        issue(tabT, tailT, full_blocks, lo, slab0, sem0)

        def pairloop(t2, fcs):
            blk0 = lo + 2 * t2
            blk1 = blk0 + 1

            @pl.when(blk1 < hi)
            def _():
                issue(tabT, tailT, full_blocks, blk1, slab1, sem1)

            drain(slab0, sem0)
            fcs = proc(blk0 - lo, slab0, fcs)

            @pl.when(blk0 + 2 < hi)
            def _():
                issue(tabT, tailT, full_blocks, blk0 + 2, slab0, sem0)

            def do1(f):
                drain(slab1, sem1)
                return proc(blk1 - lo, slab1, f)

            return lax.cond(blk1 < hi, do1, lambda f: f, fcs)

        return lax.fori_loop(0, (nb + 1) // 2, pairloop, fcs0)

    def proc_user(t, slabref, fcu):
        return ent_section(slabref, ent_u, off_u, t, fbu, du, su, fcu)

    def proc_item(t, slabref, fcs):
        fci, fcj = fcs
        fci = ent_section(slabref, ent_i, off_i, t, fbi, di, si, fci)
        fcj = ent_section(slabref, ent_j, off_j, t, fbj, dj, sj, fcj)
        return (fci, fcj)

    run_blocks(utabT, utailT, US_FULL, us_lo, us_hi, slab, slabB,
               sem0, sem1, proc_user, _i32(0))
    run_blocks(itabT, itailT, IT_FULL, it_lo, it_hi, slab, slabB,
               sem0, sem1, proc_item, (_i32(0), _i32(0)))

    flush(fbu, du, su)
    flush(fbi, di, si)
    flush(fbj, dj, sj)


def _phase2(su, si, sj, out_i, out_j, ur0, vir0, vjr0, ur1, vir1, vjr1,
            oi, oj, semA, semB):
    wid = lax.axis_index("s") * NC + lax.axis_index("c")
    b0 = wid * BPW
    lane = lax.iota(_i32, L)
    last = lane == (L - 1)
    bufs = [(ur0, vir0, vjr0, semA), (ur1, vir1, vjr1, semB)]
    NK = BPW // DP

    def issue(k, bset):
        u, vi, vj, sem = bset
        pltpu.async_copy(su.at[pl.ds(b0 + k * DP, DP)], u, sem)
        pltpu.async_copy(si.at[pl.ds(b0 + k * DP, DP)], vi, sem)
        pltpu.async_copy(sj.at[pl.ds(b0 + k * DP, DP)], vj, sem)

    def wait(bset):
        u, vi, vj, sem = bset
        for d in (u, vi, vj):
            pltpu.make_async_copy(su.at[pl.ds(0, DP)], d, sem).wait()

    issue(0, bufs[0])
    for k in range(NK):
        if k + 1 < NK:
            issue(k + 1, bufs[(k + 1) & 1])
        wait(bufs[k & 1])
        ur, vir, vjr, _ = bufs[k & 1]

        def row(r2, carry):
            for h in range(2):
                r = r2 * 2 + h
                acc_i = jnp.zeros((L,), jnp.float32)
                acc_j = jnp.zeros((L,), jnp.float32)
                for cc in range(D // L):
                    u = ur[r, pl.ds(cc * L, L)]
                    vi = vir[r, pl.ds(cc * L, L)]
                    vj = vjr[r, pl.ds(cc * L, L)]
                    acc_i = acc_i + u * vi
                    acc_j = acc_j + u * vj
                gidx = jnp.full((L,), k * DP + r, _i32)
                plsc.store_scatter(oi, [gidx], plsc.cumsum(acc_i), mask=last)
                plsc.store_scatter(oj, [gidx], plsc.cumsum(acc_j), mask=last)
            return carry

        lax.fori_loop(0, DP // 2, row, 0)

    pltpu.sync_copy(oi, out_i.at[pl.ds(b0, BPW)])
    pltpu.sync_copy(oj, out_j.at[pl.ds(b0, BPW)])


def kernel(user, item_i, item_j, embed_user_weight, embed_item_weight):
    mesh = plsc.VectorSubcoreMesh(core_axis_name="c", subcore_axis_name="s")
    cp = pltpu.CompilerParams(
        needs_layout_passes=False, use_tc_tiling_on_sc=True)

    run1 = pl.kernel(
        _phase1,
        mesh=mesh,
        compiler_params=cp,
        out_type=(
            jax.ShapeDtypeStruct((SCR, DP), jnp.float32),
            jax.ShapeDtypeStruct((SCR, DP), jnp.float32),
            jax.ShapeDtypeStruct((SCR, DP), jnp.float32),
        ),
        scratch_types=[
            pltpu.VMEM((SEG,), _i32),
            pltpu.VMEM((128,), _i32),
            pltpu.VMEM((128,), _i32),
            pltpu.VMEM((128,), _i32),
            pltpu.VMEM((128,), _i32),
            pltpu.VMEM((128,), _i32),
            pltpu.VMEM((128,), _i32),
            pltpu.VMEM((16,), _i32),
            pltpu.VMEM((16,), _i32),
            pltpu.VMEM((16,), _i32),
            pltpu.VMEM((B + L,), _i32),
            pltpu.VMEM((B + L,), _i32),
            pltpu.VMEM((B + L,), _i32),
            pltpu.VMEM((64, BW), jnp.float32),
            pltpu.VMEM((64, BW), jnp.float32),
            pltpu.VMEM((FB, DP), jnp.float32),
            pltpu.VMEM((FB, DP), jnp.float32),
            pltpu.VMEM((FB, DP), jnp.float32),
            pltpu.VMEM((FB,), _i32),
            pltpu.VMEM((FB,), _i32),
            pltpu.VMEM((FB,), _i32),
            pltpu.SemaphoreType.DMA,
            pltpu.SemaphoreType.DMA,
            pltpu.SemaphoreType.DMA,
        ],
    )

    run2 = pl.kernel(
        _phase2,
        mesh=mesh,
        compiler_params=cp,
        out_type=(
            jax.ShapeDtypeStruct((B,), jnp.float32),
            jax.ShapeDtypeStruct((B,), jnp.float32),
        ),
        scratch_types=[
            pltpu.VMEM((DP, DP), jnp.float32),
            pltpu.VMEM((DP, DP), jnp.float32),
            pltpu.VMEM((DP, DP), jnp.float32),
            pltpu.VMEM((DP, DP), jnp.float32),
            pltpu.VMEM((DP, DP), jnp.float32),
            pltpu.VMEM((DP, DP), jnp.float32),
            pltpu.VMEM((BPW,), jnp.float32),
            pltpu.VMEM((BPW,), jnp.float32),
            pltpu.SemaphoreType.DMA,
            pltpu.SemaphoreType.DMA,
        ],
    )

    u = user.astype(_i32)
    i = item_i.astype(_i32)
    j = item_j.astype(_i32)
    utabT = embed_user_weight.T
    itabT = embed_item_weight.T
    utailT = jnp.pad(embed_user_weight[US_FULL * BW:],
                     ((0, BW - (USER_N - US_FULL * BW)), (0, 0))).T
    itailT = jnp.pad(embed_item_weight[IT_FULL * BW:],
                     ((0, BW - (ITEM_N - IT_FULL * BW)), (0, 0))).T
    su, si, sj = run1(u, i, j, utabT, itabT, utailT, itailT)
    return run2(su, si, sj)
